# Initial kernel scaffold; baseline (speedup 1.0000x reference)
import jax
import jax.numpy as jnp
from jax.experimental import pallas as pl


def kernel(x, edge_index, edge_type, w1, root1, b1, w2, root2, b2, w3, root3, b3, nc_w1, nc_b1, nc_w2, nc_b2, ec_w1, ec_b1, ec_w2, ec_b2, ec_w3, ec_b3):
    raise NotImplementedError("write your pallas kernel here")


# speedup vs baseline: 13.0280x; 13.0280x over previous
import dataclasses
import functools

import jax
import jax.numpy as jnp
from jax import lax
from jax.experimental import pallas as pl
from jax.experimental.pallas import tpu as pltpu
from jax.experimental.pallas import tpu_sc as plsc

N = 10000
E = 320000
R = 3
IN = 128
H = 64

NP = 10240
NSEG = NP * R
NC, NS, L = 2, 16, 16
NW = NC * NS
CHUNK = 128
NCH = 80
EPT = NCH * CHUNK
EP = NW * EPT
SPT = NSEG // NS
NPT = NP // NS

_mesh = plsc.VectorSubcoreMesh(core_axis_name="c", subcore_axis_name="s")
f32 = jnp.float32

_sc_params = pltpu.CompilerParams()
if "needs_layout_passes" in pltpu.CompilerParams.__dataclass_fields__:
    _sc_params = dataclasses.replace(_sc_params, needs_layout_passes=False)
if "use_tc_tiling_on_sc" in pltpu.CompilerParams.__dataclass_fields__:
    _sc_params = dataclasses.replace(_sc_params, use_tc_tiling_on_sc=False)


def _hist_body(seg_hbm, hist_hbm, seg_v, hist_v):
    c = lax.axis_index("c")
    s = lax.axis_index("s")
    wid = c * NS + s
    pltpu.sync_copy(seg_hbm.at[pl.ds(wid * EPT, EPT)], seg_v)
    zeros = jnp.zeros((L,), f32)
    ones = jnp.ones((L,), f32)

    @pl.loop(0, NSEG, step=L)
    def _(i):
        hist_v[pl.ds(i, L)] = zeros

    @pl.loop(0, EPT, step=L)
    def _(i):
        idx = seg_v[pl.ds(i, L)]
        plsc.addupdate_scatter(hist_v, [idx], ones)

    pltpu.sync_copy(hist_v, hist_hbm.at[wid])


def _sc_hist(seg_flat):
    k = pl.kernel(
        _hist_body,
        out_type=jax.ShapeDtypeStruct((NW, NSEG), f32),
        mesh=_mesh,
        scratch_types=[
            pltpu.VMEM((EPT,), jnp.int32),
            pltpu.VMEM((NSEG,), f32),
        ],
        compiler_params=_sc_params,
    )
    return k(seg_flat)


_WCH = 1024


def _wts_body(rcp_hbm, seg_hbm, we_hbm, rcp_v, seg_v, we_v):
    c = lax.axis_index("c")
    s = lax.axis_index("s")
    wid = c * NS + s
    pltpu.sync_copy(rcp_hbm, rcp_v)

    @pl.loop(0, EPT // _WCH)
    def _(j):
        base = wid * EPT + j * _WCH
        pltpu.sync_copy(seg_hbm.at[pl.ds(base, _WCH)], seg_v)

        @pl.loop(0, _WCH, step=L)
        def _(i):
            idx = seg_v[pl.ds(i, L)]
            we_v[pl.ds(i, L)] = plsc.load_gather(rcp_v, [idx])

        pltpu.sync_copy(we_v, we_hbm.at[pl.ds(base, _WCH)])


def _sc_weights(rcp_flat, seg_flat):
    k = pl.kernel(
        _wts_body,
        out_type=jax.ShapeDtypeStruct((EP,), f32),
        mesh=_mesh,
        scratch_types=[
            pltpu.VMEM((NSEG,), f32),
            pltpu.VMEM((_WCH,), jnp.int32),
            pltpu.VMEM((_WCH,), f32),
        ],
        compiler_params=_sc_params,
    )
    return k(rcp_flat, seg_flat)


def _layer_body(xw_hbm, gidx_hbm, dst_hbm, we_hbm, out_hbm,
                gidx_v, dst_v, we_v, rows_v, zb_v,
                acc, sem_g, sem_s):
    c = lax.axis_index("c")
    s = lax.axis_index("s")
    wid = c * NS + s
    pltpu.sync_copy(gidx_hbm.at[wid], gidx_v)
    pltpu.sync_copy(dst_hbm.at[wid], dst_v)
    pltpu.sync_copy(we_hbm.at[wid], we_v)

    zeros = jnp.zeros((L,), f32)

    @pl.loop(0, CHUNK)
    def _(r):
        for g in range(H // L):
            zb_v[r, pl.ds(g * L, L)] = zeros

    @pl.loop(0, NPT, step=CHUNK)
    def _(k):
        pltpu.sync_copy(zb_v, acc.at[pl.ds(s * NPT + k, CHUNK)])

    plsc.subcore_barrier()

    @pl.loop(0, NCH)
    def _(j):
        pltpu.async_copy(xw_hbm.at[gidx_v.at[j]], rows_v, sem_g).wait()
        for k in range(CHUNK // L):
            wv = we_v[j, pl.ds(k * L, L)]
            for i in range(L):
                w = wv[i]
                r = k * L + i
                for g in range(H // L):
                    sl = pl.ds(g * L, L)
                    rows_v[r, sl] = w * rows_v[r, sl]
        pltpu.async_copy(rows_v, acc.at[dst_v.at[j]], sem_s, add=True).wait()

    plsc.subcore_barrier()

    @pl.loop(0, NPT, step=CHUNK)
    def _(k):
        pltpu.sync_copy(acc.at[pl.ds(s * NPT + k, CHUNK)],
                        out_hbm.at[c, pl.ds(s * NPT + k, CHUNK)])


def _sc_layer(xw_flat, gidx3, dst3, we3):
    k = pl.kernel(
        _layer_body,
        out_type=jax.ShapeDtypeStruct((NC, NP, H), f32),
        mesh=_mesh,
        scratch_types=[
            pltpu.VMEM((NCH, CHUNK), jnp.int32),
            pltpu.VMEM((NCH, CHUNK), jnp.int32),
            pltpu.VMEM((NCH, CHUNK), f32),
            pltpu.VMEM((CHUNK, H), f32),
            pltpu.VMEM((CHUNK, H), f32),
            pltpu.VMEM_SHARED((NP, H), f32),
            pltpu.SemaphoreType.DMA,
            pltpu.SemaphoreType.DMA,
        ],
        compiler_params=_sc_params,
    )
    return k(xw_flat, gidx3, dst3, we3)


def _edge_body(p_hbm, q_hbm, row_hbm, col_hbm, b1_hbm, out_hbm,
               row_v, col_v, a_v, b_v, bias_v, sem_a, sem_b):
    c = lax.axis_index("c")
    s = lax.axis_index("s")
    wid = c * NS + s
    pltpu.sync_copy(row_hbm.at[wid], row_v)
    pltpu.sync_copy(col_hbm.at[wid], col_v)
    pltpu.sync_copy(b1_hbm, bias_v)

    @pl.loop(0, NCH)
    def _(j):
        pltpu.async_copy(p_hbm.at[row_v.at[j]], a_v, sem_a).wait()
        pltpu.async_copy(q_hbm.at[col_v.at[j]], b_v, sem_b).wait()

        @pl.loop(0, CHUNK)
        def _(n):
            for g in range(H // L):
                sl = pl.ds(g * L, L)
                a_v[n, sl] = jnp.maximum(
                    a_v[n, sl] + b_v[n, sl] + bias_v[sl], 0.0)

        pltpu.sync_copy(a_v, out_hbm.at[wid, j])


def _sc_edge(p, q, row3, col3, b1):
    k = pl.kernel(
        _edge_body,
        out_type=jax.ShapeDtypeStruct((NW, NCH, CHUNK, H), f32),
        mesh=_mesh,
        scratch_types=[
            pltpu.VMEM((NCH, CHUNK), jnp.int32),
            pltpu.VMEM((NCH, CHUNK), jnp.int32),
            pltpu.VMEM((CHUNK, H), f32),
            pltpu.VMEM((CHUNK, H), f32),
            pltpu.VMEM((H,), f32),
            pltpu.SemaphoreType.DMA,
            pltpu.SemaphoreType.DMA,
        ],
        compiler_params=_sc_params,
    )
    return k(p, q, row3, col3, b1)


_BLK = 1024
_GRID = NP // _BLK


def _mm_body(x_ref, w_ref, root_ref, b_ref, xw_ref, rt_ref):
    xb = x_ref[...]
    for r in range(R):
        xw_ref[r] = jnp.dot(xb, w_ref[r], preferred_element_type=f32)
    rt_ref[...] = jnp.dot(xb, root_ref[...],
                          preferred_element_type=f32) + b_ref[...]


def _tc_mm(x, w, root, b, d_in):
    return pl.pallas_call(
        _mm_body,
        grid=(_GRID,),
        in_specs=[
            pl.BlockSpec((_BLK, d_in), lambda i: (i, 0)),
            pl.BlockSpec((R, d_in, H), lambda i: (0, 0, 0)),
            pl.BlockSpec((d_in, H), lambda i: (0, 0)),
            pl.BlockSpec((1, H), lambda i: (0, 0)),
        ],
        out_specs=[
            pl.BlockSpec((R, _BLK, H), lambda i: (0, i, 0)),
            pl.BlockSpec((_BLK, H), lambda i: (i, 0)),
        ],
        out_shape=[
            jax.ShapeDtypeStruct((R, NP, H), f32),
            jax.ShapeDtypeStruct((NP, H), f32),
        ],
    )(x, w, root, b.reshape(1, H))


def _recip_body(h_ref, o_ref):
    cnt = jnp.sum(h_ref[...], axis=0)
    rcp = 1.0 / jnp.maximum(cnt, 1.0)
    rows = jax.lax.broadcasted_iota(jnp.int32, (NSEG // 128, 128), 0)
    cols = jax.lax.broadcasted_iota(jnp.int32, (NSEG // 128, 128), 1)
    flat = rows * 128 + cols
    o_ref[...] = jnp.where(flat < N * R, rcp, 0.0)


def _tc_recip(hist):
    h3 = hist.reshape(NW, NSEG // 128, 128)
    return pl.pallas_call(
        _recip_body,
        grid=(1,),
        in_specs=[pl.BlockSpec((NW, NSEG // 128, 128), lambda i: (0, 0, 0))],
        out_specs=pl.BlockSpec((NSEG // 128, 128), lambda i: (0, 0)),
        out_shape=jax.ShapeDtypeStruct((NSEG // 128, 128), f32),
    )(h3).reshape(NSEG)


def _comb_mm_body(p_ref, rt_ref, w_ref, root_ref, b_ref,
                  xn_ref, xw_ref, rtn_ref):
    y = jnp.maximum(p_ref[0] + p_ref[1] + rt_ref[...], 0.0)
    xn_ref[...] = y
    for r in range(R):
        xw_ref[r] = jnp.dot(y, w_ref[r], preferred_element_type=f32)
    rtn_ref[...] = jnp.dot(y, root_ref[...],
                           preferred_element_type=f32) + b_ref[...]


def _tc_comb_mm(part, rt, w, root, b):
    return pl.pallas_call(
        _comb_mm_body,
        grid=(_GRID,),
        in_specs=[
            pl.BlockSpec((NC, _BLK, H), lambda i: (0, i, 0)),
            pl.BlockSpec((_BLK, H), lambda i: (i, 0)),
            pl.BlockSpec((R, H, H), lambda i: (0, 0, 0)),
            pl.BlockSpec((H, H), lambda i: (0, 0)),
            pl.BlockSpec((1, H), lambda i: (0, 0)),
        ],
        out_specs=[
            pl.BlockSpec((_BLK, H), lambda i: (i, 0)),
            pl.BlockSpec((R, _BLK, H), lambda i: (0, i, 0)),
            pl.BlockSpec((_BLK, H), lambda i: (i, 0)),
        ],
        out_shape=[
            jax.ShapeDtypeStruct((NP, H), f32),
            jax.ShapeDtypeStruct((R, NP, H), f32),
            jax.ShapeDtypeStruct((NP, H), f32),
        ],
    )(part, rt, w, root, b.reshape(1, H))


def _final_body(p_ref, rt_ref, x1_ref, ecw1_ref,
                ncw1_ref, ncb1_ref, ncw2_ref, ncb2_ref,
                pp_ref, qq_ref, no_ref):
    x3 = p_ref[0] + p_ref[1] + rt_ref[...] + x1_ref[...]
    pp_ref[...] = jnp.dot(x3, ecw1_ref[0:H], preferred_element_type=f32)
    qq_ref[...] = jnp.dot(x3, ecw1_ref[H:2 * H], preferred_element_type=f32)
    hh = jnp.maximum(
        jnp.dot(x3, ncw1_ref[...], preferred_element_type=f32)
        + ncb1_ref[...], 0.0)
    no_ref[...] = jnp.dot(hh, ncw2_ref[...],
                          preferred_element_type=f32) + ncb2_ref[...]


def _tc_final(part, rt, x1, ec_w1, nc_w1, nc_b1, nc_w2, nc_b2):
    return pl.pallas_call(
        _final_body,
        grid=(_GRID,),
        in_specs=[
            pl.BlockSpec((NC, _BLK, H), lambda i: (0, i, 0)),
            pl.BlockSpec((_BLK, H), lambda i: (i, 0)),
            pl.BlockSpec((_BLK, H), lambda i: (i, 0)),
            pl.BlockSpec((2 * H, H), lambda i: (0, 0)),
            pl.BlockSpec((H, H // 2), lambda i: (0, 0)),
            pl.BlockSpec((1, H // 2), lambda i: (0, 0)),
            pl.BlockSpec((H // 2, 2), lambda i: (0, 0)),
            pl.BlockSpec((1, 2), lambda i: (0, 0)),
        ],
        out_specs=[
            pl.BlockSpec((_BLK, H), lambda i: (i, 0)),
            pl.BlockSpec((_BLK, H), lambda i: (i, 0)),
            pl.BlockSpec((_BLK, 2), lambda i: (i, 0)),
        ],
        out_shape=[
            jax.ShapeDtypeStruct((NP, H), f32),
            jax.ShapeDtypeStruct((NP, H), f32),
            jax.ShapeDtypeStruct((NP, 2), f32),
        ],
    )(part, rt, x1, ec_w1, nc_w1, nc_b1.reshape(1, H // 2),
      nc_w2, nc_b2.reshape(1, 2))


_EBLK = 2048


def _emlp_body(e1_ref, w2_ref, b2_ref, w3_ref, b3_ref, o_ref):
    e2 = jnp.maximum(
        jnp.dot(e1_ref[...], w2_ref[...], preferred_element_type=f32)
        + b2_ref[...], 0.0)
    o_ref[...] = jnp.dot(e2, w3_ref[...],
                         preferred_element_type=f32) + b3_ref[...]


def _tc_emlp(e1, ec_w2, ec_b2, ec_w3, ec_b3):
    return pl.pallas_call(
        _emlp_body,
        grid=(EP // _EBLK,),
        in_specs=[
            pl.BlockSpec((_EBLK, H), lambda i: (i, 0)),
            pl.BlockSpec((H, H // 2), lambda i: (0, 0)),
            pl.BlockSpec((1, H // 2), lambda i: (0, 0)),
            pl.BlockSpec((H // 2, 3), lambda i: (0, 0)),
            pl.BlockSpec((1, 3), lambda i: (0, 0)),
        ],
        out_specs=pl.BlockSpec((_EBLK, 3), lambda i: (i, 0)),
        out_shape=jax.ShapeDtypeStruct((EP, 3), f32),
    )(e1, ec_w2, ec_b2.reshape(1, H // 2), ec_w3, ec_b3.reshape(1, 3))


@jax.jit
def _run(x, edge_index, edge_type,
         w1, root1, b1, w2, root2, b2, w3, root3, b3,
         nc_w1, nc_b1, nc_w2, nc_b2,
         ec_w1, ec_b1, ec_w2, ec_b2, ec_w3, ec_b3):
    src = edge_index[0]
    dst = edge_index[1]
    pad = EP - E
    ar = jnp.arange(pad, dtype=jnp.int32)
    src_p = jnp.concatenate([src, ar % NP]).astype(jnp.int32)
    col_p = jnp.concatenate([dst, ar % NP]).astype(jnp.int32)
    rel_p = jnp.concatenate([edge_type, ar % R]).astype(jnp.int32)
    seg = jnp.concatenate([dst * R + edge_type,
                           N * R + (ar % (NSEG - N * R))]).astype(jnp.int32)
    gidx = rel_p * NP + src_p

    seg_flat = seg
    gidx3 = gidx.reshape(NW, NCH, CHUNK)
    row3 = src_p.reshape(NW, NCH, CHUNK)
    col3 = col_p.reshape(NW, NCH, CHUNK)

    xpad = jnp.zeros((NP, IN), f32).at[:N].set(x)

    hist = _sc_hist(seg_flat)
    rcp = _tc_recip(hist)
    we = _sc_weights(rcp, seg_flat)
    we3 = we.reshape(NW, NCH, CHUNK)

    xw1, rt1 = _tc_mm(xpad, w1, root1, b1, IN)
    part1 = _sc_layer(xw1.reshape(R * NP, H), gidx3, col3, we3)

    x1, xw2, rt2 = _tc_comb_mm(part1, rt1, w2, root2, b2)
    part2 = _sc_layer(xw2.reshape(R * NP, H), gidx3, col3, we3)

    _, xw3, rt3 = _tc_comb_mm(part2, rt2, w3, root3, b3)
    part3 = _sc_layer(xw3.reshape(R * NP, H), gidx3, col3, we3)

    pp, qq, node_out = _tc_final(part3, rt3, x1, ec_w1,
                                 nc_w1, nc_b1, nc_w2, nc_b2)

    e1 = _sc_edge(pp, qq, row3, col3, ec_b1)
    edge_out = _tc_emlp(e1.reshape(EP, H), ec_w2, ec_b2, ec_w3, ec_b3)

    return node_out[:N], edge_out[:E]


def kernel(x, edge_index, edge_type,
           w1, root1, b1, w2, root2, b2, w3, root3, b3,
           nc_w1, nc_b1, nc_w2, nc_b2,
           ec_w1, ec_b1, ec_w2, ec_b2, ec_w3, ec_b3):
    return _run(x, edge_index, edge_type,
                w1, root1, b1, w2, root2, b2, w3, root3, b3,
                nc_w1, nc_b1, nc_w2, nc_b2,
                ec_w1, ec_b1, ec_w2, ec_b2, ec_w3, ec_b3)


# speedup vs baseline: 15.7648x; 1.2101x over previous
import dataclasses
import functools

import jax
import jax.numpy as jnp
from jax import lax
from jax.experimental import pallas as pl
from jax.experimental.pallas import tpu as pltpu
from jax.experimental.pallas import tpu_sc as plsc

N = 10000
E = 320000
R = 3
IN = 128
H = 64

NP = 10240
NSEG = NP * R
NC, NS, L = 2, 16, 16
NW = NC * NS
CHUNK = 128
NCH = 80
EPT = NCH * CHUNK
EP = NW * EPT
SPT = NSEG // NS
NPT = NP // NS

_mesh = plsc.VectorSubcoreMesh(core_axis_name="c", subcore_axis_name="s")
f32 = jnp.float32

_sc_params = pltpu.CompilerParams()
if "needs_layout_passes" in pltpu.CompilerParams.__dataclass_fields__:
    _sc_params = dataclasses.replace(_sc_params, needs_layout_passes=False)
if "use_tc_tiling_on_sc" in pltpu.CompilerParams.__dataclass_fields__:
    _sc_params = dataclasses.replace(_sc_params, use_tc_tiling_on_sc=False)


def _hist_body(seg_hbm, hist_hbm, seg_v, hist_v):
    c = lax.axis_index("c")
    s = lax.axis_index("s")
    wid = c * NS + s
    pltpu.sync_copy(seg_hbm.at[pl.ds(wid * EPT, EPT)], seg_v)
    zeros = jnp.zeros((L,), f32)
    ones = jnp.ones((L,), f32)

    @pl.loop(0, NSEG, step=L)
    def _(i):
        hist_v[pl.ds(i, L)] = zeros

    @pl.loop(0, EPT, step=L)
    def _(i):
        idx = seg_v[pl.ds(i, L)]
        plsc.addupdate_scatter(hist_v, [idx], ones)

    pltpu.sync_copy(hist_v, hist_hbm.at[wid])


def _sc_hist(seg_flat):
    k = pl.kernel(
        _hist_body,
        out_type=jax.ShapeDtypeStruct((NW, NSEG), f32),
        mesh=_mesh,
        scratch_types=[
            pltpu.VMEM((EPT,), jnp.int32),
            pltpu.VMEM((NSEG,), f32),
        ],
        compiler_params=_sc_params,
    )
    return k(seg_flat)


_WCH = 1024


def _wts_body(rcp_hbm, seg_hbm, we_hbm, rcp_v, seg_v, we_v):
    c = lax.axis_index("c")
    s = lax.axis_index("s")
    wid = c * NS + s
    pltpu.sync_copy(rcp_hbm, rcp_v)

    @pl.loop(0, EPT // _WCH)
    def _(j):
        base = wid * EPT + j * _WCH
        pltpu.sync_copy(seg_hbm.at[pl.ds(base, _WCH)], seg_v)

        @pl.loop(0, _WCH, step=L)
        def _(i):
            idx = seg_v[pl.ds(i, L)]
            we_v[pl.ds(i, L)] = plsc.load_gather(rcp_v, [idx])

        pltpu.sync_copy(we_v, we_hbm.at[pl.ds(base, _WCH)])


def _sc_weights(rcp_flat, seg_flat):
    k = pl.kernel(
        _wts_body,
        out_type=jax.ShapeDtypeStruct((EP,), f32),
        mesh=_mesh,
        scratch_types=[
            pltpu.VMEM((NSEG,), f32),
            pltpu.VMEM((_WCH,), jnp.int32),
            pltpu.VMEM((_WCH,), f32),
        ],
        compiler_params=_sc_params,
    )
    return k(rcp_flat, seg_flat)


_NBUF = 4


def _wmul(rows, we_v, ch):
    for k in range(CHUNK // L):
        wv = we_v[ch, pl.ds(k * L, L)]
        for i in range(L):
            w = wv[i]
            r = k * L + i
            for g in range(H // L):
                sl = pl.ds(g * L, L)
                rows[r, sl] = w * rows[r, sl]


def _layer_body(xw_hbm, gidx_hbm, dst_hbm, we_hbm, out_hbm,
                gidx_v, dst_v, we_v, r0, r1, r2, r3, zb_v, acc,
                sg0, sg1, sg2, sg3, ss0, ss1, ss2, ss3):
    rows = (r0, r1, r2, r3)
    sgs = (sg0, sg1, sg2, sg3)
    sss = (ss0, ss1, ss2, ss3)
    c = lax.axis_index("c")
    s = lax.axis_index("s")
    wid = c * NS + s
    pltpu.sync_copy(gidx_hbm.at[wid], gidx_v)
    pltpu.sync_copy(dst_hbm.at[wid], dst_v)
    pltpu.sync_copy(we_hbm.at[wid], we_v)

    zeros = jnp.zeros((L,), f32)

    @pl.loop(0, CHUNK)
    def _(r):
        for g in range(H // L):
            zb_v[r, pl.ds(g * L, L)] = zeros

    @pl.loop(0, NPT, step=CHUNK)
    def _(k):
        pltpu.sync_copy(zb_v, acc.at[pl.ds(s * NPT + k, CHUNK)])

    plsc.subcore_barrier()

    for b in range(_NBUF):
        pltpu.async_copy(xw_hbm.at[gidx_v.at[b]], rows[b], sgs[b])

    @pl.loop(0, NCH - _NBUF, step=_NBUF)
    def _(j):
        for b in range(_NBUF):
            ch = j + b
            pltpu.make_async_copy(xw_hbm.at[gidx_v.at[b]], rows[b],
                                  sgs[b]).wait()
            _wmul(rows[b], we_v, ch)
            pltpu.async_copy(rows[b], acc.at[dst_v.at[ch]], sss[b],
                             add=True).wait()
            pltpu.async_copy(xw_hbm.at[gidx_v.at[ch + _NBUF]], rows[b],
                             sgs[b])

    for b in range(_NBUF):
        ch = NCH - _NBUF + b
        pltpu.make_async_copy(xw_hbm.at[gidx_v.at[b]], rows[b], sgs[b]).wait()
        _wmul(rows[b], we_v, ch)
        pltpu.async_copy(rows[b], acc.at[dst_v.at[ch]], sss[b],
                         add=True).wait()

    plsc.subcore_barrier()

    @pl.loop(0, NPT, step=CHUNK)
    def _(k):
        pltpu.sync_copy(acc.at[pl.ds(s * NPT + k, CHUNK)],
                        out_hbm.at[c, pl.ds(s * NPT + k, CHUNK)])


def _sc_layer(xw_flat, gidx3, dst3, we3):
    k = pl.kernel(
        _layer_body,
        out_type=jax.ShapeDtypeStruct((NC, NP, H), f32),
        mesh=_mesh,
        scratch_types=[
            pltpu.VMEM((NCH, CHUNK), jnp.int32),
            pltpu.VMEM((NCH, CHUNK), jnp.int32),
            pltpu.VMEM((NCH, CHUNK), f32),
            pltpu.VMEM((CHUNK, H), f32),
            pltpu.VMEM((CHUNK, H), f32),
            pltpu.VMEM((CHUNK, H), f32),
            pltpu.VMEM((CHUNK, H), f32),
            pltpu.VMEM((CHUNK, H), f32),
            pltpu.VMEM_SHARED((NP, H), f32),
            pltpu.SemaphoreType.DMA,
            pltpu.SemaphoreType.DMA,
            pltpu.SemaphoreType.DMA,
            pltpu.SemaphoreType.DMA,
            pltpu.SemaphoreType.DMA,
            pltpu.SemaphoreType.DMA,
            pltpu.SemaphoreType.DMA,
            pltpu.SemaphoreType.DMA,
        ],
        compiler_params=_sc_params,
    )
    return k(xw_flat, gidx3, dst3, we3)


def _eadd(a, b, bias_v):
    @pl.loop(0, CHUNK)
    def _(n):
        for g in range(H // L):
            sl = pl.ds(g * L, L)
            a[n, sl] = jnp.maximum(a[n, sl] + b[n, sl] + bias_v[sl], 0.0)


def _edge_body(p_hbm, q_hbm, row_hbm, col_hbm, b1_hbm, out_hbm,
               row_v, col_v, a0, a1, b0, b1v, bias_v,
               sa0, sa1, sb0, sb1, so0, so1):
    av = (a0, a1)
    bv = (b0, b1v)
    sas = (sa0, sa1)
    sbs = (sb0, sb1)
    sos = (so0, so1)
    c = lax.axis_index("c")
    s = lax.axis_index("s")
    wid = c * NS + s
    pltpu.sync_copy(row_hbm.at[wid], row_v)
    pltpu.sync_copy(col_hbm.at[wid], col_v)
    pltpu.sync_copy(b1_hbm, bias_v)

    for b in range(2):
        pltpu.async_copy(p_hbm.at[row_v.at[b]], av[b], sas[b])
        pltpu.async_copy(q_hbm.at[col_v.at[b]], bv[b], sbs[b])

    @pl.loop(0, NCH - 2, step=2)
    def _(j):
        for b in range(2):
            ch = j + b
            pltpu.make_async_copy(p_hbm.at[row_v.at[b]], av[b], sas[b]).wait()
            pltpu.make_async_copy(q_hbm.at[col_v.at[b]], bv[b], sbs[b]).wait()
            _eadd(av[b], bv[b], bias_v)
            pltpu.async_copy(av[b], out_hbm.at[wid, ch], sos[b])
            pltpu.async_copy(q_hbm.at[col_v.at[ch + 2]], bv[b], sbs[b])
            pltpu.make_async_copy(av[b], out_hbm.at[wid, ch], sos[b]).wait()
            pltpu.async_copy(p_hbm.at[row_v.at[ch + 2]], av[b], sas[b])

    for b in range(2):
        ch = NCH - 2 + b
        pltpu.make_async_copy(p_hbm.at[row_v.at[b]], av[b], sas[b]).wait()
        pltpu.make_async_copy(q_hbm.at[col_v.at[b]], bv[b], sbs[b]).wait()
        _eadd(av[b], bv[b], bias_v)
        pltpu.async_copy(av[b], out_hbm.at[wid, ch], sos[b]).wait()


def _sc_edge(p, q, row3, col3, b1):
    k = pl.kernel(
        _edge_body,
        out_type=jax.ShapeDtypeStruct((NW, NCH, CHUNK, H), f32),
        mesh=_mesh,
        scratch_types=[
            pltpu.VMEM((NCH, CHUNK), jnp.int32),
            pltpu.VMEM((NCH, CHUNK), jnp.int32),
            pltpu.VMEM((CHUNK, H), f32),
            pltpu.VMEM((CHUNK, H), f32),
            pltpu.VMEM((CHUNK, H), f32),
            pltpu.VMEM((CHUNK, H), f32),
            pltpu.VMEM((H,), f32),
            pltpu.SemaphoreType.DMA,
            pltpu.SemaphoreType.DMA,
            pltpu.SemaphoreType.DMA,
            pltpu.SemaphoreType.DMA,
            pltpu.SemaphoreType.DMA,
            pltpu.SemaphoreType.DMA,
        ],
        compiler_params=_sc_params,
    )
    return k(p, q, row3, col3, b1)


_BLK = 1024
_GRID = NP // _BLK


def _mm_body(x_ref, w_ref, root_ref, b_ref, xw_ref, rt_ref):
    xb = x_ref[...]
    for r in range(R):
        xw_ref[r] = jnp.dot(xb, w_ref[r], preferred_element_type=f32)
    rt_ref[...] = jnp.dot(xb, root_ref[...],
                          preferred_element_type=f32) + b_ref[...]


def _tc_mm(x, w, root, b, d_in):
    return pl.pallas_call(
        _mm_body,
        grid=(_GRID,),
        in_specs=[
            pl.BlockSpec((_BLK, d_in), lambda i: (i, 0)),
            pl.BlockSpec((R, d_in, H), lambda i: (0, 0, 0)),
            pl.BlockSpec((d_in, H), lambda i: (0, 0)),
            pl.BlockSpec((1, H), lambda i: (0, 0)),
        ],
        out_specs=[
            pl.BlockSpec((R, _BLK, H), lambda i: (0, i, 0)),
            pl.BlockSpec((_BLK, H), lambda i: (i, 0)),
        ],
        out_shape=[
            jax.ShapeDtypeStruct((R, NP, H), f32),
            jax.ShapeDtypeStruct((NP, H), f32),
        ],
    )(x, w, root, b.reshape(1, H))


def _recip_body(h_ref, o_ref):
    cnt = jnp.sum(h_ref[...], axis=0)
    rcp = 1.0 / jnp.maximum(cnt, 1.0)
    rows = jax.lax.broadcasted_iota(jnp.int32, (NSEG // 128, 128), 0)
    cols = jax.lax.broadcasted_iota(jnp.int32, (NSEG // 128, 128), 1)
    flat = rows * 128 + cols
    o_ref[...] = jnp.where(flat < N * R, rcp, 0.0)


def _tc_recip(hist):
    h3 = hist.reshape(NW, NSEG // 128, 128)
    return pl.pallas_call(
        _recip_body,
        grid=(1,),
        in_specs=[pl.BlockSpec((NW, NSEG // 128, 128), lambda i: (0, 0, 0))],
        out_specs=pl.BlockSpec((NSEG // 128, 128), lambda i: (0, 0)),
        out_shape=jax.ShapeDtypeStruct((NSEG // 128, 128), f32),
    )(h3).reshape(NSEG)


def _comb_mm_body(p_ref, rt_ref, w_ref, root_ref, b_ref,
                  xn_ref, xw_ref, rtn_ref):
    y = jnp.maximum(p_ref[0] + p_ref[1] + rt_ref[...], 0.0)
    xn_ref[...] = y
    for r in range(R):
        xw_ref[r] = jnp.dot(y, w_ref[r], preferred_element_type=f32)
    rtn_ref[...] = jnp.dot(y, root_ref[...],
                           preferred_element_type=f32) + b_ref[...]


def _tc_comb_mm(part, rt, w, root, b):
    return pl.pallas_call(
        _comb_mm_body,
        grid=(_GRID,),
        in_specs=[
            pl.BlockSpec((NC, _BLK, H), lambda i: (0, i, 0)),
            pl.BlockSpec((_BLK, H), lambda i: (i, 0)),
            pl.BlockSpec((R, H, H), lambda i: (0, 0, 0)),
            pl.BlockSpec((H, H), lambda i: (0, 0)),
            pl.BlockSpec((1, H), lambda i: (0, 0)),
        ],
        out_specs=[
            pl.BlockSpec((_BLK, H), lambda i: (i, 0)),
            pl.BlockSpec((R, _BLK, H), lambda i: (0, i, 0)),
            pl.BlockSpec((_BLK, H), lambda i: (i, 0)),
        ],
        out_shape=[
            jax.ShapeDtypeStruct((NP, H), f32),
            jax.ShapeDtypeStruct((R, NP, H), f32),
            jax.ShapeDtypeStruct((NP, H), f32),
        ],
    )(part, rt, w, root, b.reshape(1, H))


def _final_body(p_ref, rt_ref, x1_ref, ecw1_ref,
                ncw1_ref, ncb1_ref, ncw2_ref, ncb2_ref,
                pp_ref, qq_ref, no_ref):
    x3 = p_ref[0] + p_ref[1] + rt_ref[...] + x1_ref[...]
    pp_ref[...] = jnp.dot(x3, ecw1_ref[0:H], preferred_element_type=f32)
    qq_ref[...] = jnp.dot(x3, ecw1_ref[H:2 * H], preferred_element_type=f32)
    hh = jnp.maximum(
        jnp.dot(x3, ncw1_ref[...], preferred_element_type=f32)
        + ncb1_ref[...], 0.0)
    no_ref[...] = jnp.dot(hh, ncw2_ref[...],
                          preferred_element_type=f32) + ncb2_ref[...]


def _tc_final(part, rt, x1, ec_w1, nc_w1, nc_b1, nc_w2, nc_b2):
    return pl.pallas_call(
        _final_body,
        grid=(_GRID,),
        in_specs=[
            pl.BlockSpec((NC, _BLK, H), lambda i: (0, i, 0)),
            pl.BlockSpec((_BLK, H), lambda i: (i, 0)),
            pl.BlockSpec((_BLK, H), lambda i: (i, 0)),
            pl.BlockSpec((2 * H, H), lambda i: (0, 0)),
            pl.BlockSpec((H, H // 2), lambda i: (0, 0)),
            pl.BlockSpec((1, H // 2), lambda i: (0, 0)),
            pl.BlockSpec((H // 2, 2), lambda i: (0, 0)),
            pl.BlockSpec((1, 2), lambda i: (0, 0)),
        ],
        out_specs=[
            pl.BlockSpec((_BLK, H), lambda i: (i, 0)),
            pl.BlockSpec((_BLK, H), lambda i: (i, 0)),
            pl.BlockSpec((_BLK, 2), lambda i: (i, 0)),
        ],
        out_shape=[
            jax.ShapeDtypeStruct((NP, H), f32),
            jax.ShapeDtypeStruct((NP, H), f32),
            jax.ShapeDtypeStruct((NP, 2), f32),
        ],
    )(part, rt, x1, ec_w1, nc_w1, nc_b1.reshape(1, H // 2),
      nc_w2, nc_b2.reshape(1, 2))


_EBLK = 2048


def _emlp_body(e1_ref, w2_ref, b2_ref, w3_ref, b3_ref, o_ref):
    e2 = jnp.maximum(
        jnp.dot(e1_ref[...], w2_ref[...], preferred_element_type=f32)
        + b2_ref[...], 0.0)
    o_ref[...] = jnp.dot(e2, w3_ref[...],
                         preferred_element_type=f32) + b3_ref[...]


def _tc_emlp(e1, ec_w2, ec_b2, ec_w3, ec_b3):
    return pl.pallas_call(
        _emlp_body,
        grid=(EP // _EBLK,),
        in_specs=[
            pl.BlockSpec((_EBLK, H), lambda i: (i, 0)),
            pl.BlockSpec((H, H // 2), lambda i: (0, 0)),
            pl.BlockSpec((1, H // 2), lambda i: (0, 0)),
            pl.BlockSpec((H // 2, 3), lambda i: (0, 0)),
            pl.BlockSpec((1, 3), lambda i: (0, 0)),
        ],
        out_specs=pl.BlockSpec((_EBLK, 3), lambda i: (i, 0)),
        out_shape=jax.ShapeDtypeStruct((EP, 3), f32),
    )(e1, ec_w2, ec_b2.reshape(1, H // 2), ec_w3, ec_b3.reshape(1, 3))


@jax.jit
def _run(x, edge_index, edge_type,
         w1, root1, b1, w2, root2, b2, w3, root3, b3,
         nc_w1, nc_b1, nc_w2, nc_b2,
         ec_w1, ec_b1, ec_w2, ec_b2, ec_w3, ec_b3):
    src = edge_index[0]
    dst = edge_index[1]
    pad = EP - E
    ar = jnp.arange(pad, dtype=jnp.int32)
    src_p = jnp.concatenate([src, ar % NP]).astype(jnp.int32)
    col_p = jnp.concatenate([dst, ar % NP]).astype(jnp.int32)
    rel_p = jnp.concatenate([edge_type, ar % R]).astype(jnp.int32)
    seg = jnp.concatenate([dst * R + edge_type,
                           N * R + (ar % (NSEG - N * R))]).astype(jnp.int32)
    gidx = rel_p * NP + src_p

    seg_flat = seg
    gidx3 = gidx.reshape(NW, NCH, CHUNK)
    row3 = src_p.reshape(NW, NCH, CHUNK)
    col3 = col_p.reshape(NW, NCH, CHUNK)

    xpad = jnp.zeros((NP, IN), f32).at[:N].set(x)

    hist = _sc_hist(seg_flat)
    rcp = _tc_recip(hist)
    we = _sc_weights(rcp, seg_flat)
    we3 = we.reshape(NW, NCH, CHUNK)

    xw1, rt1 = _tc_mm(xpad, w1, root1, b1, IN)
    part1 = _sc_layer(xw1.reshape(R * NP, H), gidx3, col3, we3)

    x1, xw2, rt2 = _tc_comb_mm(part1, rt1, w2, root2, b2)
    part2 = _sc_layer(xw2.reshape(R * NP, H), gidx3, col3, we3)

    _, xw3, rt3 = _tc_comb_mm(part2, rt2, w3, root3, b3)
    part3 = _sc_layer(xw3.reshape(R * NP, H), gidx3, col3, we3)

    pp, qq, node_out = _tc_final(part3, rt3, x1, ec_w1,
                                 nc_w1, nc_b1, nc_w2, nc_b2)

    e1 = _sc_edge(pp, qq, row3, col3, ec_b1)
    edge_out = _tc_emlp(e1.reshape(EP, H), ec_w2, ec_b2, ec_w3, ec_b3)

    return node_out[:N], edge_out[:E]


def kernel(x, edge_index, edge_type,
           w1, root1, b1, w2, root2, b2, w3, root3, b3,
           nc_w1, nc_b1, nc_w2, nc_b2,
           ec_w1, ec_b1, ec_w2, ec_b2, ec_w3, ec_b3):
    return _run(x, edge_index, edge_type,
                w1, root1, b1, w2, root2, b2, w3, root3, b3,
                nc_w1, nc_b1, nc_w2, nc_b2,
                ec_w1, ec_b1, ec_w2, ec_b2, ec_w3, ec_b3)


# speedup vs baseline: 16.4879x; 1.0459x over previous
import dataclasses
import functools

import jax
import jax.numpy as jnp
from jax import lax
from jax.experimental import pallas as pl
from jax.experimental.pallas import tpu as pltpu
from jax.experimental.pallas import tpu_sc as plsc

N = 10000
E = 320000
R = 3
IN = 128
H = 64

NP = 10240
NSEG = NP * R
NC, NS, L = 2, 16, 16
NW = NC * NS
CHUNK = 128
NCH = 80
EPT = NCH * CHUNK
EP = NW * EPT
SPT = NSEG // NS
NPT = NP // NS

_mesh = plsc.VectorSubcoreMesh(core_axis_name="c", subcore_axis_name="s")
f32 = jnp.float32

_sc_params = pltpu.CompilerParams()
if "needs_layout_passes" in pltpu.CompilerParams.__dataclass_fields__:
    _sc_params = dataclasses.replace(_sc_params, needs_layout_passes=False)
if "use_tc_tiling_on_sc" in pltpu.CompilerParams.__dataclass_fields__:
    _sc_params = dataclasses.replace(_sc_params, use_tc_tiling_on_sc=False)


def _hist_body(seg_hbm, hist_hbm, seg_v, hist_v):
    c = lax.axis_index("c")
    s = lax.axis_index("s")
    wid = c * NS + s
    pltpu.sync_copy(seg_hbm.at[pl.ds(wid * EPT, EPT)], seg_v)
    zeros = jnp.zeros((L,), f32)
    ones = jnp.ones((L,), f32)

    @pl.loop(0, NSEG, step=L)
    def _(i):
        hist_v[pl.ds(i, L)] = zeros

    @pl.loop(0, EPT, step=L)
    def _(i):
        idx = seg_v[pl.ds(i, L)]
        plsc.addupdate_scatter(hist_v, [idx], ones)

    pltpu.sync_copy(hist_v, hist_hbm.at[wid])


def _sc_hist(seg_flat):
    k = pl.kernel(
        _hist_body,
        out_type=jax.ShapeDtypeStruct((NW, NSEG), f32),
        mesh=_mesh,
        scratch_types=[
            pltpu.VMEM((EPT,), jnp.int32),
            pltpu.VMEM((NSEG,), f32),
        ],
        compiler_params=_sc_params,
    )
    return k(seg_flat)


_WCH = 1024


def _wts_body(rcp_hbm, seg_hbm, we_hbm, rcp_v, seg_v, we_v):
    c = lax.axis_index("c")
    s = lax.axis_index("s")
    wid = c * NS + s
    pltpu.sync_copy(rcp_hbm, rcp_v)

    @pl.loop(0, EPT // _WCH)
    def _(j):
        base = wid * EPT + j * _WCH
        pltpu.sync_copy(seg_hbm.at[pl.ds(base, _WCH)], seg_v)

        @pl.loop(0, _WCH, step=L)
        def _(i):
            idx = seg_v[pl.ds(i, L)]
            we_v[pl.ds(i, L)] = plsc.load_gather(rcp_v, [idx])

        pltpu.sync_copy(we_v, we_hbm.at[pl.ds(base, _WCH)])


def _sc_weights(rcp_flat, seg_flat):
    k = pl.kernel(
        _wts_body,
        out_type=jax.ShapeDtypeStruct((EP,), f32),
        mesh=_mesh,
        scratch_types=[
            pltpu.VMEM((NSEG,), f32),
            pltpu.VMEM((_WCH,), jnp.int32),
            pltpu.VMEM((_WCH,), f32),
        ],
        compiler_params=_sc_params,
    )
    return k(rcp_flat, seg_flat)


_NBUF = 4


def _wmul(rows, we_v, ch):
    for k in range(CHUNK // L):
        wv = we_v[ch, pl.ds(k * L, L)]
        for i in range(L):
            w = wv[i]
            r = k * L + i
            for g in range(H // L):
                sl = pl.ds(g * L, L)
                rows[r, sl] = w * rows[r, sl]


def _layer_body(xw_hbm, gidx_hbm, dst_hbm, we_hbm, out_hbm,
                gidx_v, dst_v, we_v, r0, r1, r2, r3, zb_v, acc,
                sg0, sg1, sg2, sg3, ss0, ss1, ss2, ss3):
    rows = (r0, r1, r2, r3)
    sgs = (sg0, sg1, sg2, sg3)
    sss = (ss0, ss1, ss2, ss3)
    c = lax.axis_index("c")
    s = lax.axis_index("s")
    wid = c * NS + s
    pltpu.sync_copy(gidx_hbm.at[wid], gidx_v)
    pltpu.sync_copy(dst_hbm.at[wid], dst_v)
    pltpu.sync_copy(we_hbm.at[wid], we_v)

    zeros = jnp.zeros((L,), f32)

    @pl.loop(0, CHUNK)
    def _(r):
        for g in range(H // L):
            zb_v[r, pl.ds(g * L, L)] = zeros

    @pl.loop(0, NPT, step=CHUNK)
    def _(k):
        pltpu.sync_copy(zb_v, acc.at[pl.ds(s * NPT + k, CHUNK)])

    plsc.subcore_barrier()

    def wait_g(b):
        pltpu.make_async_copy(xw_hbm.at[gidx_v.at[b]], rows[b],
                              sgs[b]).wait()

    def issue_s(ch, b):
        pltpu.async_copy(rows[b], acc.at[dst_v.at[ch]], sss[b], add=True)

    def wait_s(b):
        pltpu.make_async_copy(rows[b], acc.at[dst_v.at[0]], sss[b]).wait()

    def issue_g(ch, b):
        pltpu.async_copy(xw_hbm.at[gidx_v.at[ch]], rows[b], sgs[b])

    for b in range(3):
        issue_g(b, b)
    wait_g(0)
    _wmul(rows[0], we_v, 0)
    issue_s(0, 0)
    issue_g(3, 3)
    for ch in range(1, _NBUF):
        b = ch
        wait_g(b)
        _wmul(rows[b], we_v, ch)
        issue_s(ch, b)
        wait_s(b - 1)
        issue_g(ch + 3, (ch + 3) % _NBUF)

    @pl.loop(_NBUF, NCH - _NBUF, step=_NBUF)
    def _(j):
        for b in range(_NBUF):
            ch = j + b
            bo = (b + 3) % _NBUF
            wait_g(b)
            _wmul(rows[b], we_v, ch)
            issue_s(ch, b)
            wait_s(bo)
            issue_g(ch + 3, bo)

    for ch in range(NCH - _NBUF, NCH):
        b = ch % _NBUF
        bo = (b + 3) % _NBUF
        wait_g(b)
        _wmul(rows[b], we_v, ch)
        issue_s(ch, b)
        wait_s(bo)
        if ch + 3 < NCH:
            issue_g(ch + 3, bo)
    wait_s((NCH - 1) % _NBUF)

    plsc.subcore_barrier()

    @pl.loop(0, NPT, step=CHUNK)
    def _(k):
        pltpu.sync_copy(acc.at[pl.ds(s * NPT + k, CHUNK)],
                        out_hbm.at[c, pl.ds(s * NPT + k, CHUNK)])


def _sc_layer(xw_flat, gidx3, dst3, we3):
    k = pl.kernel(
        _layer_body,
        out_type=jax.ShapeDtypeStruct((NC, NP, H), f32),
        mesh=_mesh,
        scratch_types=[
            pltpu.VMEM((NCH, CHUNK), jnp.int32),
            pltpu.VMEM((NCH, CHUNK), jnp.int32),
            pltpu.VMEM((NCH, CHUNK), f32),
            pltpu.VMEM((CHUNK, H), f32),
            pltpu.VMEM((CHUNK, H), f32),
            pltpu.VMEM((CHUNK, H), f32),
            pltpu.VMEM((CHUNK, H), f32),
            pltpu.VMEM((CHUNK, H), f32),
            pltpu.VMEM_SHARED((NP, H), f32),
            pltpu.SemaphoreType.DMA,
            pltpu.SemaphoreType.DMA,
            pltpu.SemaphoreType.DMA,
            pltpu.SemaphoreType.DMA,
            pltpu.SemaphoreType.DMA,
            pltpu.SemaphoreType.DMA,
            pltpu.SemaphoreType.DMA,
            pltpu.SemaphoreType.DMA,
        ],
        compiler_params=_sc_params,
    )
    return k(xw_flat, gidx3, dst3, we3)


def _eadd(a, b, bias_v):
    @pl.loop(0, CHUNK)
    def _(n):
        for g in range(H // L):
            sl = pl.ds(g * L, L)
            a[n, sl] = jnp.maximum(a[n, sl] + b[n, sl] + bias_v[sl], 0.0)


def _edge_body(p_hbm, q_hbm, row_hbm, col_hbm, b1_hbm, out_hbm,
               row_v, col_v, a0, a1, a2, a3, b0, b1v, b2, b3, bias_v,
               sa0, sa1, sa2, sa3, sb0, sb1, sb2, sb3,
               so0, so1, so2, so3):
    av = (a0, a1, a2, a3)
    bv = (b0, b1v, b2, b3)
    sas = (sa0, sa1, sa2, sa3)
    sbs = (sb0, sb1, sb2, sb3)
    sos = (so0, so1, so2, so3)
    c = lax.axis_index("c")
    s = lax.axis_index("s")
    wid = c * NS + s
    pltpu.sync_copy(row_hbm.at[wid], row_v)
    pltpu.sync_copy(col_hbm.at[wid], col_v)
    pltpu.sync_copy(b1_hbm, bias_v)

    def wait_pq(b):
        pltpu.make_async_copy(p_hbm.at[row_v.at[b]], av[b], sas[b]).wait()
        pltpu.make_async_copy(q_hbm.at[col_v.at[b]], bv[b], sbs[b]).wait()

    def issue_o(ch, b):
        pltpu.async_copy(av[b], out_hbm.at[wid, ch], sos[b])

    def wait_o(b):
        pltpu.make_async_copy(av[b], out_hbm.at[wid, 0], sos[b]).wait()

    def issue_pq(ch, b):
        pltpu.async_copy(p_hbm.at[row_v.at[ch]], av[b], sas[b])
        pltpu.async_copy(q_hbm.at[col_v.at[ch]], bv[b], sbs[b])

    for b in range(3):
        issue_pq(b, b)
    wait_pq(0)
    _eadd(av[0], bv[0], bias_v)
    issue_o(0, 0)
    issue_pq(3, 3)
    for ch in range(1, _NBUF):
        b = ch
        wait_pq(b)
        _eadd(av[b], bv[b], bias_v)
        issue_o(ch, b)
        wait_o(b - 1)
        issue_pq(ch + 3, (ch + 3) % _NBUF)

    @pl.loop(_NBUF, NCH - _NBUF, step=_NBUF)
    def _(j):
        for b in range(_NBUF):
            ch = j + b
            bo = (b + 3) % _NBUF
            wait_pq(b)
            _eadd(av[b], bv[b], bias_v)
            issue_o(ch, b)
            wait_o(bo)
            issue_pq(ch + 3, bo)

    for ch in range(NCH - _NBUF, NCH):
        b = ch % _NBUF
        bo = (b + 3) % _NBUF
        wait_pq(b)
        _eadd(av[b], bv[b], bias_v)
        issue_o(ch, b)
        wait_o(bo)
        if ch + 3 < NCH:
            issue_pq(ch + 3, bo)
    wait_o((NCH - 1) % _NBUF)


def _sc_edge(p, q, row3, col3, b1):
    k = pl.kernel(
        _edge_body,
        out_type=jax.ShapeDtypeStruct((NW, NCH, CHUNK, H), f32),
        mesh=_mesh,
        scratch_types=(
            [pltpu.VMEM((NCH, CHUNK), jnp.int32)] * 2
            + [pltpu.VMEM((CHUNK, H), f32)] * 8
            + [pltpu.VMEM((H,), f32)]
            + [pltpu.SemaphoreType.DMA] * 12
        ),
        compiler_params=_sc_params,
    )
    return k(p, q, row3, col3, b1)


_BLK = 1024
_GRID = NP // _BLK


def _mm_body(x_ref, w_ref, root_ref, b_ref, xw_ref, rt_ref):
    xb = x_ref[...]
    for r in range(R):
        xw_ref[r] = jnp.dot(xb, w_ref[r], preferred_element_type=f32)
    rt_ref[...] = jnp.dot(xb, root_ref[...],
                          preferred_element_type=f32) + b_ref[...]


def _tc_mm(x, w, root, b, d_in):
    return pl.pallas_call(
        _mm_body,
        grid=(_GRID,),
        in_specs=[
            pl.BlockSpec((_BLK, d_in), lambda i: (i, 0)),
            pl.BlockSpec((R, d_in, H), lambda i: (0, 0, 0)),
            pl.BlockSpec((d_in, H), lambda i: (0, 0)),
            pl.BlockSpec((1, H), lambda i: (0, 0)),
        ],
        out_specs=[
            pl.BlockSpec((R, _BLK, H), lambda i: (0, i, 0)),
            pl.BlockSpec((_BLK, H), lambda i: (i, 0)),
        ],
        out_shape=[
            jax.ShapeDtypeStruct((R, NP, H), f32),
            jax.ShapeDtypeStruct((NP, H), f32),
        ],
    )(x, w, root, b.reshape(1, H))


def _recip_body(h_ref, o_ref):
    cnt = jnp.sum(h_ref[...], axis=0)
    rcp = 1.0 / jnp.maximum(cnt, 1.0)
    rows = jax.lax.broadcasted_iota(jnp.int32, (NSEG // 128, 128), 0)
    cols = jax.lax.broadcasted_iota(jnp.int32, (NSEG // 128, 128), 1)
    flat = rows * 128 + cols
    o_ref[...] = jnp.where(flat < N * R, rcp, 0.0)


def _tc_recip(hist):
    h3 = hist.reshape(NW, NSEG // 128, 128)
    return pl.pallas_call(
        _recip_body,
        grid=(1,),
        in_specs=[pl.BlockSpec((NW, NSEG // 128, 128), lambda i: (0, 0, 0))],
        out_specs=pl.BlockSpec((NSEG // 128, 128), lambda i: (0, 0)),
        out_shape=jax.ShapeDtypeStruct((NSEG // 128, 128), f32),
    )(h3).reshape(NSEG)


def _comb_mm_body(p_ref, rt_ref, w_ref, root_ref, b_ref,
                  xn_ref, xw_ref, rtn_ref):
    y = jnp.maximum(p_ref[0] + p_ref[1] + rt_ref[...], 0.0)
    xn_ref[...] = y
    for r in range(R):
        xw_ref[r] = jnp.dot(y, w_ref[r], preferred_element_type=f32)
    rtn_ref[...] = jnp.dot(y, root_ref[...],
                           preferred_element_type=f32) + b_ref[...]


def _tc_comb_mm(part, rt, w, root, b):
    return pl.pallas_call(
        _comb_mm_body,
        grid=(_GRID,),
        in_specs=[
            pl.BlockSpec((NC, _BLK, H), lambda i: (0, i, 0)),
            pl.BlockSpec((_BLK, H), lambda i: (i, 0)),
            pl.BlockSpec((R, H, H), lambda i: (0, 0, 0)),
            pl.BlockSpec((H, H), lambda i: (0, 0)),
            pl.BlockSpec((1, H), lambda i: (0, 0)),
        ],
        out_specs=[
            pl.BlockSpec((_BLK, H), lambda i: (i, 0)),
            pl.BlockSpec((R, _BLK, H), lambda i: (0, i, 0)),
            pl.BlockSpec((_BLK, H), lambda i: (i, 0)),
        ],
        out_shape=[
            jax.ShapeDtypeStruct((NP, H), f32),
            jax.ShapeDtypeStruct((R, NP, H), f32),
            jax.ShapeDtypeStruct((NP, H), f32),
        ],
    )(part, rt, w, root, b.reshape(1, H))


def _final_body(p_ref, rt_ref, x1_ref, ecw1_ref,
                ncw1_ref, ncb1_ref, ncw2_ref, ncb2_ref,
                pp_ref, qq_ref, no_ref):
    x3 = p_ref[0] + p_ref[1] + rt_ref[...] + x1_ref[...]
    pp_ref[...] = jnp.dot(x3, ecw1_ref[0:H], preferred_element_type=f32)
    qq_ref[...] = jnp.dot(x3, ecw1_ref[H:2 * H], preferred_element_type=f32)
    hh = jnp.maximum(
        jnp.dot(x3, ncw1_ref[...], preferred_element_type=f32)
        + ncb1_ref[...], 0.0)
    no_ref[...] = jnp.dot(hh, ncw2_ref[...],
                          preferred_element_type=f32) + ncb2_ref[...]


def _tc_final(part, rt, x1, ec_w1, nc_w1, nc_b1, nc_w2, nc_b2):
    return pl.pallas_call(
        _final_body,
        grid=(_GRID,),
        in_specs=[
            pl.BlockSpec((NC, _BLK, H), lambda i: (0, i, 0)),
            pl.BlockSpec((_BLK, H), lambda i: (i, 0)),
            pl.BlockSpec((_BLK, H), lambda i: (i, 0)),
            pl.BlockSpec((2 * H, H), lambda i: (0, 0)),
            pl.BlockSpec((H, H // 2), lambda i: (0, 0)),
            pl.BlockSpec((1, H // 2), lambda i: (0, 0)),
            pl.BlockSpec((H // 2, 2), lambda i: (0, 0)),
            pl.BlockSpec((1, 2), lambda i: (0, 0)),
        ],
        out_specs=[
            pl.BlockSpec((_BLK, H), lambda i: (i, 0)),
            pl.BlockSpec((_BLK, H), lambda i: (i, 0)),
            pl.BlockSpec((_BLK, 2), lambda i: (i, 0)),
        ],
        out_shape=[
            jax.ShapeDtypeStruct((NP, H), f32),
            jax.ShapeDtypeStruct((NP, H), f32),
            jax.ShapeDtypeStruct((NP, 2), f32),
        ],
    )(part, rt, x1, ec_w1, nc_w1, nc_b1.reshape(1, H // 2),
      nc_w2, nc_b2.reshape(1, 2))


_EBLK = 2048


def _emlp_body(e1_ref, w2_ref, b2_ref, w3_ref, b3_ref, o_ref):
    e2 = jnp.maximum(
        jnp.dot(e1_ref[...], w2_ref[...], preferred_element_type=f32)
        + b2_ref[...], 0.0)
    o_ref[...] = jnp.dot(e2, w3_ref[...],
                         preferred_element_type=f32) + b3_ref[...]


def _tc_emlp(e1, ec_w2, ec_b2, ec_w3, ec_b3):
    return pl.pallas_call(
        _emlp_body,
        grid=(EP // _EBLK,),
        in_specs=[
            pl.BlockSpec((_EBLK, H), lambda i: (i, 0)),
            pl.BlockSpec((H, H // 2), lambda i: (0, 0)),
            pl.BlockSpec((1, H // 2), lambda i: (0, 0)),
            pl.BlockSpec((H // 2, 3), lambda i: (0, 0)),
            pl.BlockSpec((1, 3), lambda i: (0, 0)),
        ],
        out_specs=pl.BlockSpec((_EBLK, 3), lambda i: (i, 0)),
        out_shape=jax.ShapeDtypeStruct((EP, 3), f32),
    )(e1, ec_w2, ec_b2.reshape(1, H // 2), ec_w3, ec_b3.reshape(1, 3))


@jax.jit
def _run(x, edge_index, edge_type,
         w1, root1, b1, w2, root2, b2, w3, root3, b3,
         nc_w1, nc_b1, nc_w2, nc_b2,
         ec_w1, ec_b1, ec_w2, ec_b2, ec_w3, ec_b3):
    src = edge_index[0]
    dst = edge_index[1]
    pad = EP - E
    ar = jnp.arange(pad, dtype=jnp.int32)
    src_p = jnp.concatenate([src, ar % NP]).astype(jnp.int32)
    col_p = jnp.concatenate([dst, ar % NP]).astype(jnp.int32)
    rel_p = jnp.concatenate([edge_type, ar % R]).astype(jnp.int32)
    seg = jnp.concatenate([dst * R + edge_type,
                           N * R + (ar % (NSEG - N * R))]).astype(jnp.int32)
    gidx = rel_p * NP + src_p

    seg_flat = seg
    gidx3 = gidx.reshape(NW, NCH, CHUNK)
    row3 = src_p.reshape(NW, NCH, CHUNK)
    col3 = col_p.reshape(NW, NCH, CHUNK)

    xpad = jnp.zeros((NP, IN), f32).at[:N].set(x)

    hist = _sc_hist(seg_flat)
    rcp = _tc_recip(hist)
    we = _sc_weights(rcp, seg_flat)
    we3 = we.reshape(NW, NCH, CHUNK)

    xw1, rt1 = _tc_mm(xpad, w1, root1, b1, IN)
    part1 = _sc_layer(xw1.reshape(R * NP, H), gidx3, col3, we3)

    x1, xw2, rt2 = _tc_comb_mm(part1, rt1, w2, root2, b2)
    part2 = _sc_layer(xw2.reshape(R * NP, H), gidx3, col3, we3)

    _, xw3, rt3 = _tc_comb_mm(part2, rt2, w3, root3, b3)
    part3 = _sc_layer(xw3.reshape(R * NP, H), gidx3, col3, we3)

    pp, qq, node_out = _tc_final(part3, rt3, x1, ec_w1,
                                 nc_w1, nc_b1, nc_w2, nc_b2)

    e1 = _sc_edge(pp, qq, row3, col3, ec_b1)
    edge_out = _tc_emlp(e1.reshape(EP, H), ec_w2, ec_b2, ec_w3, ec_b3)

    return node_out[:N], edge_out[:E]


def kernel(x, edge_index, edge_type,
           w1, root1, b1, w2, root2, b2, w3, root3, b3,
           nc_w1, nc_b1, nc_w2, nc_b2,
           ec_w1, ec_b1, ec_w2, ec_b2, ec_w3, ec_b3):
    return _run(x, edge_index, edge_type,
                w1, root1, b1, w2, root2, b2, w3, root3, b3,
                nc_w1, nc_b1, nc_w2, nc_b2,
                ec_w1, ec_b1, ec_w2, ec_b2, ec_w3, ec_b3)


# speedup vs baseline: 17.0710x; 1.0354x over previous
import dataclasses
import functools

import jax
import jax.numpy as jnp
from jax import lax
from jax.experimental import pallas as pl
from jax.experimental.pallas import tpu as pltpu
from jax.experimental.pallas import tpu_sc as plsc

N = 10000
E = 320000
R = 3
IN = 128
H = 64

NP = 10240
NSEG = NP * R
NC, NS, L = 2, 16, 16
NW = NC * NS
CHUNK = 128
NCH = 80
EPT = NCH * CHUNK
EP = NW * EPT
SPT = NSEG // NS
NPT = NP // NS

_mesh = plsc.VectorSubcoreMesh(core_axis_name="c", subcore_axis_name="s")
f32 = jnp.float32

_sc_params = pltpu.CompilerParams()
if "needs_layout_passes" in pltpu.CompilerParams.__dataclass_fields__:
    _sc_params = dataclasses.replace(_sc_params, needs_layout_passes=False)
if "use_tc_tiling_on_sc" in pltpu.CompilerParams.__dataclass_fields__:
    _sc_params = dataclasses.replace(_sc_params, use_tc_tiling_on_sc=False)


def _hist_body(seg_hbm, hist_hbm, seg_v, hist_v):
    c = lax.axis_index("c")
    s = lax.axis_index("s")
    wid = c * NS + s
    pltpu.sync_copy(seg_hbm.at[pl.ds(wid * EPT, EPT)], seg_v)
    zeros = jnp.zeros((L,), f32)
    ones = jnp.ones((L,), f32)

    @pl.loop(0, NSEG, step=L)
    def _(i):
        hist_v[pl.ds(i, L)] = zeros

    @pl.loop(0, EPT, step=L)
    def _(i):
        idx = seg_v[pl.ds(i, L)]
        plsc.addupdate_scatter(hist_v, [idx], ones)

    pltpu.sync_copy(hist_v, hist_hbm.at[wid])


def _sc_hist(seg_flat):
    k = pl.kernel(
        _hist_body,
        out_type=jax.ShapeDtypeStruct((NW, NSEG), f32),
        mesh=_mesh,
        scratch_types=[
            pltpu.VMEM((EPT,), jnp.int32),
            pltpu.VMEM((NSEG,), f32),
        ],
        compiler_params=_sc_params,
    )
    return k(seg_flat)


_WCH = 1024


def _wts_body(rcp_hbm, seg_hbm, we_hbm, rcp_v, seg_v, we_v):
    c = lax.axis_index("c")
    s = lax.axis_index("s")
    wid = c * NS + s
    pltpu.sync_copy(rcp_hbm, rcp_v)

    @pl.loop(0, EPT // _WCH)
    def _(j):
        base = wid * EPT + j * _WCH
        pltpu.sync_copy(seg_hbm.at[pl.ds(base, _WCH)], seg_v)

        @pl.loop(0, _WCH, step=L)
        def _(i):
            idx = seg_v[pl.ds(i, L)]
            we_v[pl.ds(i, L)] = plsc.load_gather(rcp_v, [idx])

        pltpu.sync_copy(we_v, we_hbm.at[pl.ds(base, _WCH)])


def _sc_weights(rcp_flat, seg_flat):
    k = pl.kernel(
        _wts_body,
        out_type=jax.ShapeDtypeStruct((EP,), f32),
        mesh=_mesh,
        scratch_types=[
            pltpu.VMEM((NSEG,), f32),
            pltpu.VMEM((_WCH,), jnp.int32),
            pltpu.VMEM((_WCH,), f32),
        ],
        compiler_params=_sc_params,
    )
    return k(rcp_flat, seg_flat)


_NBUF = 4


def _wmul(rows, we_v, ch):
    for k in range(CHUNK // L):
        wv = we_v[ch, pl.ds(k * L, L)]
        for i in range(L):
            w = wv[i]
            r = k * L + i
            for g in range(H // L):
                sl = pl.ds(g * L, L)
                rows[r, sl] = w * rows[r, sl]


def _layer_body(xw_hbm, gidx_hbm, dst_hbm, we_hbm, out_hbm,
                gidx_v, dst_v, we_v, r0, r1, r2, r3, zb_v, acc,
                sg0, sg1, sg2, sg3, ss0, ss1, ss2, ss3):
    rows = (r0, r1, r2, r3)
    sgs = (sg0, sg1, sg2, sg3)
    sss = (ss0, ss1, ss2, ss3)
    c = lax.axis_index("c")
    s = lax.axis_index("s")
    wid = c * NS + s
    pltpu.sync_copy(gidx_hbm.at[wid], gidx_v)
    pltpu.sync_copy(dst_hbm.at[wid], dst_v)
    pltpu.sync_copy(we_hbm.at[wid], we_v)

    zeros = jnp.zeros((L,), f32)

    @pl.loop(0, CHUNK)
    def _(r):
        for g in range(H // L):
            zb_v[r, pl.ds(g * L, L)] = zeros

    @pl.loop(0, NPT, step=CHUNK)
    def _(k):
        pltpu.sync_copy(zb_v, acc.at[pl.ds(s * NPT + k, CHUNK)])

    plsc.subcore_barrier()

    def wait_g(b):
        pltpu.make_async_copy(xw_hbm.at[gidx_v.at[b]], rows[b],
                              sgs[b]).wait()

    def issue_s(ch, b):
        pltpu.async_copy(rows[b], acc.at[dst_v.at[ch]], sss[b], add=True)

    def wait_s(b):
        pltpu.make_async_copy(rows[b], acc.at[dst_v.at[0]], sss[b]).wait()

    def issue_g(ch, b):
        pltpu.async_copy(xw_hbm.at[gidx_v.at[ch]], rows[b], sgs[b])

    for b in range(3):
        issue_g(b, b)
    wait_g(0)
    _wmul(rows[0], we_v, 0)
    issue_s(0, 0)
    issue_g(3, 3)
    for ch in range(1, _NBUF):
        b = ch
        wait_g(b)
        _wmul(rows[b], we_v, ch)
        issue_s(ch, b)
        wait_s(b - 1)
        issue_g(ch + 3, (ch + 3) % _NBUF)

    @pl.loop(_NBUF, NCH - _NBUF, step=_NBUF)
    def _(j):
        for b in range(_NBUF):
            ch = j + b
            bo = (b + 3) % _NBUF
            wait_g(b)
            _wmul(rows[b], we_v, ch)
            issue_s(ch, b)
            wait_s(bo)
            issue_g(ch + 3, bo)

    for ch in range(NCH - _NBUF, NCH):
        b = ch % _NBUF
        bo = (b + 3) % _NBUF
        wait_g(b)
        _wmul(rows[b], we_v, ch)
        issue_s(ch, b)
        wait_s(bo)
        if ch + 3 < NCH:
            issue_g(ch + 3, bo)
    wait_s((NCH - 1) % _NBUF)

    plsc.subcore_barrier()

    @pl.loop(0, NPT, step=CHUNK)
    def _(k):
        pltpu.sync_copy(acc.at[pl.ds(s * NPT + k, CHUNK)],
                        out_hbm.at[c, pl.ds(s * NPT + k, CHUNK)])


def _sc_layer(xw_flat, gidx3, dst3, we3):
    k = pl.kernel(
        _layer_body,
        out_type=jax.ShapeDtypeStruct((NC, NP, H), f32),
        mesh=_mesh,
        scratch_types=[
            pltpu.VMEM((NCH, CHUNK), jnp.int32),
            pltpu.VMEM((NCH, CHUNK), jnp.int32),
            pltpu.VMEM((NCH, CHUNK), f32),
            pltpu.VMEM((CHUNK, H), f32),
            pltpu.VMEM((CHUNK, H), f32),
            pltpu.VMEM((CHUNK, H), f32),
            pltpu.VMEM((CHUNK, H), f32),
            pltpu.VMEM((CHUNK, H), f32),
            pltpu.VMEM_SHARED((NP, H), f32),
            pltpu.SemaphoreType.DMA,
            pltpu.SemaphoreType.DMA,
            pltpu.SemaphoreType.DMA,
            pltpu.SemaphoreType.DMA,
            pltpu.SemaphoreType.DMA,
            pltpu.SemaphoreType.DMA,
            pltpu.SemaphoreType.DMA,
            pltpu.SemaphoreType.DMA,
        ],
        compiler_params=_sc_params,
    )
    return k(xw_flat, gidx3, dst3, we3)


def _eadd(a, b, bias_v):
    @pl.loop(0, CHUNK)
    def _(n):
        for g in range(H // L):
            sl = pl.ds(g * L, L)
            a[n, sl] = jnp.maximum(a[n, sl] + b[n, sl] + bias_v[sl], 0.0)


def _edge_body(p_hbm, q_hbm, row_hbm, col_hbm, b1_hbm, out_hbm,
               row_v, col_v, a0, a1, a2, a3, b0, b1v, b2, b3, bias_v,
               sa0, sa1, sa2, sa3, sb0, sb1, sb2, sb3,
               so0, so1, so2, so3):
    av = (a0, a1, a2, a3)
    bv = (b0, b1v, b2, b3)
    sas = (sa0, sa1, sa2, sa3)
    sbs = (sb0, sb1, sb2, sb3)
    sos = (so0, so1, so2, so3)
    c = lax.axis_index("c")
    s = lax.axis_index("s")
    wid = c * NS + s
    pltpu.sync_copy(row_hbm.at[wid], row_v)
    pltpu.sync_copy(col_hbm.at[wid], col_v)
    pltpu.sync_copy(b1_hbm, bias_v)

    def wait_pq(b):
        pltpu.make_async_copy(p_hbm.at[row_v.at[b]], av[b], sas[b]).wait()
        pltpu.make_async_copy(q_hbm.at[col_v.at[b]], bv[b], sbs[b]).wait()

    def issue_o(ch, b):
        pltpu.async_copy(av[b],
                         out_hbm.at[pl.ds((wid * NCH + ch) * CHUNK, CHUNK)],
                         sos[b])

    def wait_o(b):
        pltpu.make_async_copy(av[b], out_hbm.at[pl.ds(0, CHUNK)],
                              sos[b]).wait()

    def issue_pq(ch, b):
        pltpu.async_copy(p_hbm.at[row_v.at[ch]], av[b], sas[b])
        pltpu.async_copy(q_hbm.at[col_v.at[ch]], bv[b], sbs[b])

    for b in range(3):
        issue_pq(b, b)
    wait_pq(0)
    _eadd(av[0], bv[0], bias_v)
    issue_o(0, 0)
    issue_pq(3, 3)
    for ch in range(1, _NBUF):
        b = ch
        wait_pq(b)
        _eadd(av[b], bv[b], bias_v)
        issue_o(ch, b)
        wait_o(b - 1)
        issue_pq(ch + 3, (ch + 3) % _NBUF)

    @pl.loop(_NBUF, NCH - _NBUF, step=_NBUF)
    def _(j):
        for b in range(_NBUF):
            ch = j + b
            bo = (b + 3) % _NBUF
            wait_pq(b)
            _eadd(av[b], bv[b], bias_v)
            issue_o(ch, b)
            wait_o(bo)
            issue_pq(ch + 3, bo)

    for ch in range(NCH - _NBUF, NCH):
        b = ch % _NBUF
        bo = (b + 3) % _NBUF
        wait_pq(b)
        _eadd(av[b], bv[b], bias_v)
        issue_o(ch, b)
        wait_o(bo)
        if ch + 3 < NCH:
            issue_pq(ch + 3, bo)
    wait_o((NCH - 1) % _NBUF)


def _sc_edge(p, q, row3, col3, b1):
    k = pl.kernel(
        _edge_body,
        out_type=jax.ShapeDtypeStruct((EP, H), f32),
        mesh=_mesh,
        scratch_types=(
            [pltpu.VMEM((NCH, CHUNK), jnp.int32)] * 2
            + [pltpu.VMEM((CHUNK, H), f32)] * 8
            + [pltpu.VMEM((H,), f32)]
            + [pltpu.SemaphoreType.DMA] * 12
        ),
        compiler_params=_sc_params,
    )
    return k(p, q, row3, col3, b1)


_BLK = 1024
_GRID = NP // _BLK


def _mm_body(x_ref, w_ref, root_ref, b_ref, xw_ref, rt_ref):
    r = pl.program_id(1)
    xb = x_ref[...]
    xw_ref[...] = jnp.dot(xb, w_ref[0], preferred_element_type=f32)

    @pl.when(r == 0)
    def _():
        rt_ref[...] = jnp.dot(xb, root_ref[...],
                              preferred_element_type=f32) + b_ref[...]


def _tc_mm(x, w, root, b, d_in):
    return pl.pallas_call(
        _mm_body,
        grid=(_GRID, R),
        in_specs=[
            pl.BlockSpec((_BLK, d_in), lambda i, r: (i, 0)),
            pl.BlockSpec((1, d_in, H), lambda i, r: (r, 0, 0)),
            pl.BlockSpec((d_in, H), lambda i, r: (0, 0)),
            pl.BlockSpec((1, H), lambda i, r: (0, 0)),
        ],
        out_specs=[
            pl.BlockSpec((_BLK, H), lambda i, r: (r * _GRID + i, 0)),
            pl.BlockSpec((_BLK, H), lambda i, r: (i, 0)),
        ],
        out_shape=[
            jax.ShapeDtypeStruct((R * NP, H), f32),
            jax.ShapeDtypeStruct((NP, H), f32),
        ],
    )(x, w, root, b.reshape(1, H))


def _recip_body(h_ref, o_ref):
    cnt = jnp.sum(h_ref[...], axis=0)
    rcp = 1.0 / jnp.maximum(cnt, 1.0)
    rows = jax.lax.broadcasted_iota(jnp.int32, (NSEG // 128, 128), 0)
    cols = jax.lax.broadcasted_iota(jnp.int32, (NSEG // 128, 128), 1)
    flat = rows * 128 + cols
    o_ref[...] = jnp.where(flat < N * R, rcp, 0.0)


def _tc_recip(hist):
    h3 = hist.reshape(NW, NSEG // 128, 128)
    return pl.pallas_call(
        _recip_body,
        grid=(1,),
        in_specs=[pl.BlockSpec((NW, NSEG // 128, 128), lambda i: (0, 0, 0))],
        out_specs=pl.BlockSpec((NSEG // 128, 128), lambda i: (0, 0)),
        out_shape=jax.ShapeDtypeStruct((NSEG // 128, 128), f32),
    )(h3).reshape(NSEG)


def _comb_mm_body(p_ref, rt_ref, w_ref, root_ref, b_ref,
                  xn_ref, xw_ref, rtn_ref):
    r = pl.program_id(1)
    y = jnp.maximum(p_ref[0] + p_ref[1] + rt_ref[...], 0.0)
    xw_ref[...] = jnp.dot(y, w_ref[0], preferred_element_type=f32)

    @pl.when(r == 0)
    def _():
        xn_ref[...] = y
        rtn_ref[...] = jnp.dot(y, root_ref[...],
                               preferred_element_type=f32) + b_ref[...]


def _tc_comb_mm(part, rt, w, root, b):
    return pl.pallas_call(
        _comb_mm_body,
        grid=(_GRID, R),
        in_specs=[
            pl.BlockSpec((NC, _BLK, H), lambda i, r: (0, i, 0)),
            pl.BlockSpec((_BLK, H), lambda i, r: (i, 0)),
            pl.BlockSpec((1, H, H), lambda i, r: (r, 0, 0)),
            pl.BlockSpec((H, H), lambda i, r: (0, 0)),
            pl.BlockSpec((1, H), lambda i, r: (0, 0)),
        ],
        out_specs=[
            pl.BlockSpec((_BLK, H), lambda i, r: (i, 0)),
            pl.BlockSpec((_BLK, H), lambda i, r: (r * _GRID + i, 0)),
            pl.BlockSpec((_BLK, H), lambda i, r: (i, 0)),
        ],
        out_shape=[
            jax.ShapeDtypeStruct((NP, H), f32),
            jax.ShapeDtypeStruct((R * NP, H), f32),
            jax.ShapeDtypeStruct((NP, H), f32),
        ],
    )(part, rt, w, root, b.reshape(1, H))


def _final_body(p_ref, rt_ref, x1_ref, ecw1_ref,
                ncw1_ref, ncb1_ref, ncw2_ref, ncb2_ref,
                pp_ref, qq_ref, no_ref):
    x3 = p_ref[0] + p_ref[1] + rt_ref[...] + x1_ref[...]
    pp_ref[...] = jnp.dot(x3, ecw1_ref[0:H], preferred_element_type=f32)
    qq_ref[...] = jnp.dot(x3, ecw1_ref[H:2 * H], preferred_element_type=f32)
    hh = jnp.maximum(
        jnp.dot(x3, ncw1_ref[...], preferred_element_type=f32)
        + ncb1_ref[...], 0.0)
    no_ref[...] = jnp.dot(hh, ncw2_ref[...],
                          preferred_element_type=f32) + ncb2_ref[...]


def _tc_final(part, rt, x1, ec_w1, nc_w1, nc_b1, nc_w2, nc_b2):
    return pl.pallas_call(
        _final_body,
        grid=(_GRID,),
        in_specs=[
            pl.BlockSpec((NC, _BLK, H), lambda i: (0, i, 0)),
            pl.BlockSpec((_BLK, H), lambda i: (i, 0)),
            pl.BlockSpec((_BLK, H), lambda i: (i, 0)),
            pl.BlockSpec((2 * H, H), lambda i: (0, 0)),
            pl.BlockSpec((H, H // 2), lambda i: (0, 0)),
            pl.BlockSpec((1, H // 2), lambda i: (0, 0)),
            pl.BlockSpec((H // 2, 2), lambda i: (0, 0)),
            pl.BlockSpec((1, 2), lambda i: (0, 0)),
        ],
        out_specs=[
            pl.BlockSpec((_BLK, H), lambda i: (i, 0)),
            pl.BlockSpec((_BLK, H), lambda i: (i, 0)),
            pl.BlockSpec((_BLK, 2), lambda i: (i, 0)),
        ],
        out_shape=[
            jax.ShapeDtypeStruct((NP, H), f32),
            jax.ShapeDtypeStruct((NP, H), f32),
            jax.ShapeDtypeStruct((NP, 2), f32),
        ],
    )(part, rt, x1, ec_w1, nc_w1, nc_b1.reshape(1, H // 2),
      nc_w2, nc_b2.reshape(1, 2))


_EBLK = 1600


def _emlp_body(e1_ref, w2_ref, b2_ref, w3_ref, b3_ref, o_ref):
    e2 = jnp.maximum(
        jnp.dot(e1_ref[...], w2_ref[...], preferred_element_type=f32)
        + b2_ref[...], 0.0)
    o_ref[...] = jnp.dot(e2, w3_ref[...],
                         preferred_element_type=f32) + b3_ref[...]


def _tc_emlp(e1, ec_w2, ec_b2, ec_w3, ec_b3):
    return pl.pallas_call(
        _emlp_body,
        grid=(E // _EBLK,),
        in_specs=[
            pl.BlockSpec((_EBLK, H), lambda i: (i, 0)),
            pl.BlockSpec((H, H // 2), lambda i: (0, 0)),
            pl.BlockSpec((1, H // 2), lambda i: (0, 0)),
            pl.BlockSpec((H // 2, 3), lambda i: (0, 0)),
            pl.BlockSpec((1, 3), lambda i: (0, 0)),
        ],
        out_specs=pl.BlockSpec((_EBLK, 3), lambda i: (i, 0)),
        out_shape=jax.ShapeDtypeStruct((E, 3), f32),
    )(e1, ec_w2, ec_b2.reshape(1, H // 2), ec_w3, ec_b3.reshape(1, 3))


@jax.jit
def _run(x, edge_index, edge_type,
         w1, root1, b1, w2, root2, b2, w3, root3, b3,
         nc_w1, nc_b1, nc_w2, nc_b2,
         ec_w1, ec_b1, ec_w2, ec_b2, ec_w3, ec_b3):
    src = edge_index[0]
    dst = edge_index[1]
    pad = EP - E
    ar = jnp.arange(pad, dtype=jnp.int32)
    src_p = jnp.concatenate([src, ar % NP]).astype(jnp.int32)
    col_p = jnp.concatenate([dst, ar % NP]).astype(jnp.int32)
    rel_p = jnp.concatenate([edge_type, ar % R]).astype(jnp.int32)
    seg = jnp.concatenate([dst * R + edge_type,
                           N * R + (ar % (NSEG - N * R))]).astype(jnp.int32)
    gidx = rel_p * NP + src_p

    seg_flat = seg
    gidx3 = gidx.reshape(NW, NCH, CHUNK)
    row3 = src_p.reshape(NW, NCH, CHUNK)
    col3 = col_p.reshape(NW, NCH, CHUNK)

    xpad = jnp.zeros((NP, IN), f32).at[:N].set(x)

    hist = _sc_hist(seg_flat)
    rcp = _tc_recip(hist)
    we = _sc_weights(rcp, seg_flat)
    we3 = we.reshape(NW, NCH, CHUNK)

    xw1, rt1 = _tc_mm(xpad, w1, root1, b1, IN)
    part1 = _sc_layer(xw1, gidx3, col3, we3)

    x1, xw2, rt2 = _tc_comb_mm(part1, rt1, w2, root2, b2)
    part2 = _sc_layer(xw2, gidx3, col3, we3)

    _, xw3, rt3 = _tc_comb_mm(part2, rt2, w3, root3, b3)
    part3 = _sc_layer(xw3, gidx3, col3, we3)

    pp, qq, node_out = _tc_final(part3, rt3, x1, ec_w1,
                                 nc_w1, nc_b1, nc_w2, nc_b2)

    e1 = _sc_edge(pp, qq, row3, col3, ec_b1)
    edge_out = _tc_emlp(e1, ec_w2, ec_b2, ec_w3, ec_b3)

    return node_out[:N], edge_out


def kernel(x, edge_index, edge_type,
           w1, root1, b1, w2, root2, b2, w3, root3, b3,
           nc_w1, nc_b1, nc_w2, nc_b2,
           ec_w1, ec_b1, ec_w2, ec_b2, ec_w3, ec_b3):
    return _run(x, edge_index, edge_type,
                w1, root1, b1, w2, root2, b2, w3, root3, b3,
                nc_w1, nc_b1, nc_w2, nc_b2,
                ec_w1, ec_b1, ec_w2, ec_b2, ec_w3, ec_b3)


# speedup vs baseline: 17.7113x; 1.0375x over previous
import dataclasses
import functools

import jax
import jax.numpy as jnp
from jax import lax
from jax.experimental import pallas as pl
from jax.experimental.pallas import tpu as pltpu
from jax.experimental.pallas import tpu_sc as plsc

N = 10000
E = 320000
R = 3
IN = 128
H = 64

NP = 10240
NSEG = NP * R
NC, NS, L = 2, 16, 16
NW = NC * NS
CHUNK = 128
NCH = 80
EPT = NCH * CHUNK
EP = NW * EPT
SPT = NSEG // NS
NPT = NP // NS

_mesh = plsc.VectorSubcoreMesh(core_axis_name="c", subcore_axis_name="s")
f32 = jnp.float32

_sc_params = pltpu.CompilerParams()
if "needs_layout_passes" in pltpu.CompilerParams.__dataclass_fields__:
    _sc_params = dataclasses.replace(_sc_params, needs_layout_passes=False)
if "use_tc_tiling_on_sc" in pltpu.CompilerParams.__dataclass_fields__:
    _sc_params = dataclasses.replace(_sc_params, use_tc_tiling_on_sc=False)


def _hist_body(seg_hbm, hist_hbm, seg_v, hist_v):
    c = lax.axis_index("c")
    s = lax.axis_index("s")
    wid = c * NS + s
    pltpu.sync_copy(seg_hbm.at[pl.ds(wid * EPT, EPT)], seg_v)
    zeros = jnp.zeros((L,), f32)
    ones = jnp.ones((L,), f32)

    @pl.loop(0, NSEG, step=L)
    def _(i):
        hist_v[pl.ds(i, L)] = zeros

    @pl.loop(0, EPT, step=L)
    def _(i):
        idx = seg_v[pl.ds(i, L)]
        plsc.addupdate_scatter(hist_v, [idx], ones)

    pltpu.sync_copy(hist_v, hist_hbm.at[wid])


def _sc_hist(seg_flat):
    k = pl.kernel(
        _hist_body,
        out_type=jax.ShapeDtypeStruct((NW, NSEG), f32),
        mesh=_mesh,
        scratch_types=[
            pltpu.VMEM((EPT,), jnp.int32),
            pltpu.VMEM((NSEG,), f32),
        ],
        compiler_params=_sc_params,
    )
    return k(seg_flat)


_WCH = 1024


def _wts_body(hist_hbm, seg_hbm, we_hbm, h_v, rcp_v, seg_v, we_v, rcp_sh):
    c = lax.axis_index("c")
    s = lax.axis_index("s")
    wid = c * NS + s
    pltpu.sync_copy(hist_hbm.at[:, pl.ds(s * SPT, SPT)], h_v)

    @pl.loop(0, SPT, step=L)
    def _(i):
        cnt = h_v[0, pl.ds(i, L)]
        for t in range(1, NW):
            cnt = cnt + h_v[t, pl.ds(i, L)]
        bid = jax.lax.iota(jnp.int32, L) + (s * SPT + i)
        r = 1.0 / jnp.maximum(cnt, 1.0)
        rcp_v[pl.ds(s * SPT + i, L)] = jnp.where(bid < N * R, r, 0.0)

    pltpu.sync_copy(rcp_v.at[pl.ds(s * SPT, SPT)],
                    rcp_sh.at[pl.ds(s * SPT, SPT)])
    plsc.subcore_barrier()
    pltpu.sync_copy(rcp_sh, rcp_v)

    @pl.loop(0, EPT // _WCH)
    def _(j):
        base = wid * EPT + j * _WCH
        pltpu.sync_copy(seg_hbm.at[pl.ds(base, _WCH)], seg_v)

        @pl.loop(0, _WCH, step=L)
        def _(i):
            idx = seg_v[pl.ds(i, L)]
            we_v[pl.ds(i, L)] = plsc.load_gather(rcp_v, [idx])

        pltpu.sync_copy(we_v, we_hbm.at[pl.ds(base, _WCH)])


def _sc_weights(hist, seg_flat):
    k = pl.kernel(
        _wts_body,
        out_type=jax.ShapeDtypeStruct((EP,), f32),
        mesh=_mesh,
        scratch_types=[
            pltpu.VMEM((NW, SPT), f32),
            pltpu.VMEM((NSEG,), f32),
            pltpu.VMEM((_WCH,), jnp.int32),
            pltpu.VMEM((_WCH,), f32),
            pltpu.VMEM_SHARED((NSEG,), f32),
        ],
        compiler_params=_sc_params,
    )
    return k(hist, seg_flat)


_NBUF = 4


def _wmul(rows, we_v, ch):
    for k in range(CHUNK // L):
        wv = we_v[ch, pl.ds(k * L, L)]
        for i in range(L):
            w = wv[i]
            r = k * L + i
            for g in range(H // L):
                sl = pl.ds(g * L, L)
                rows[r, sl] = w * rows[r, sl]


def _layer_body(xw_hbm, gidx_hbm, dst_hbm, we_hbm, out_hbm,
                gidx_v, dst_v, we_v, r0, r1, r2, r3, zb_v, acc,
                sg0, sg1, sg2, sg3, ss0, ss1, ss2, ss3):
    rows = (r0, r1, r2, r3)
    sgs = (sg0, sg1, sg2, sg3)
    sss = (ss0, ss1, ss2, ss3)
    c = lax.axis_index("c")
    s = lax.axis_index("s")
    wid = c * NS + s
    pltpu.sync_copy(gidx_hbm.at[wid], gidx_v)
    pltpu.sync_copy(dst_hbm.at[wid], dst_v)
    pltpu.sync_copy(we_hbm.at[wid], we_v)

    zeros = jnp.zeros((L,), f32)

    @pl.loop(0, CHUNK)
    def _(r):
        for g in range(H // L):
            zb_v[r, pl.ds(g * L, L)] = zeros

    @pl.loop(0, NPT, step=CHUNK)
    def _(k):
        pltpu.sync_copy(zb_v, acc.at[pl.ds(s * NPT + k, CHUNK)])

    plsc.subcore_barrier()

    def wait_g(b):
        pltpu.make_async_copy(xw_hbm.at[gidx_v.at[b]], rows[b],
                              sgs[b]).wait()

    def issue_s(ch, b):
        pltpu.async_copy(rows[b], acc.at[dst_v.at[ch]], sss[b], add=True)

    def wait_s(b):
        pltpu.make_async_copy(rows[b], acc.at[dst_v.at[0]], sss[b]).wait()

    def issue_g(ch, b):
        pltpu.async_copy(xw_hbm.at[gidx_v.at[ch]], rows[b], sgs[b])

    for b in range(3):
        issue_g(b, b)
    wait_g(0)
    _wmul(rows[0], we_v, 0)
    issue_s(0, 0)
    issue_g(3, 3)
    for ch in range(1, _NBUF):
        b = ch
        wait_g(b)
        _wmul(rows[b], we_v, ch)
        issue_s(ch, b)
        wait_s(b - 1)
        issue_g(ch + 3, (ch + 3) % _NBUF)

    @pl.loop(_NBUF, NCH - _NBUF, step=_NBUF)
    def _(j):
        for b in range(_NBUF):
            ch = j + b
            bo = (b + 3) % _NBUF
            wait_g(b)
            _wmul(rows[b], we_v, ch)
            issue_s(ch, b)
            wait_s(bo)
            issue_g(ch + 3, bo)

    for ch in range(NCH - _NBUF, NCH):
        b = ch % _NBUF
        bo = (b + 3) % _NBUF
        wait_g(b)
        _wmul(rows[b], we_v, ch)
        issue_s(ch, b)
        wait_s(bo)
        if ch + 3 < NCH:
            issue_g(ch + 3, bo)
    wait_s((NCH - 1) % _NBUF)

    plsc.subcore_barrier()

    @pl.loop(0, NPT, step=CHUNK)
    def _(k):
        pltpu.sync_copy(acc.at[pl.ds(s * NPT + k, CHUNK)],
                        out_hbm.at[c, pl.ds(s * NPT + k, CHUNK)])


def _sc_layer(xw_flat, gidx3, dst3, we3):
    k = pl.kernel(
        _layer_body,
        out_type=jax.ShapeDtypeStruct((NC, NP, H), f32),
        mesh=_mesh,
        scratch_types=[
            pltpu.VMEM((NCH, CHUNK), jnp.int32),
            pltpu.VMEM((NCH, CHUNK), jnp.int32),
            pltpu.VMEM((NCH, CHUNK), f32),
            pltpu.VMEM((CHUNK, H), f32),
            pltpu.VMEM((CHUNK, H), f32),
            pltpu.VMEM((CHUNK, H), f32),
            pltpu.VMEM((CHUNK, H), f32),
            pltpu.VMEM((CHUNK, H), f32),
            pltpu.VMEM_SHARED((NP, H), f32),
            pltpu.SemaphoreType.DMA,
            pltpu.SemaphoreType.DMA,
            pltpu.SemaphoreType.DMA,
            pltpu.SemaphoreType.DMA,
            pltpu.SemaphoreType.DMA,
            pltpu.SemaphoreType.DMA,
            pltpu.SemaphoreType.DMA,
            pltpu.SemaphoreType.DMA,
        ],
        compiler_params=_sc_params,
    )
    return k(xw_flat, gidx3, dst3, we3)


bf16 = jnp.bfloat16
_L2 = 2 * L


def _eadd(a, b, bias_v):
    zero = bf16(0.0)

    @pl.loop(0, CHUNK)
    def _(n):
        for g in range(H // _L2):
            sl = pl.ds(g * _L2, _L2)
            a[n, sl] = jnp.maximum(a[n, sl] + b[n, sl] + bias_v[sl], zero)


def _edge_body(p_hbm, q_hbm, row_hbm, col_hbm, b1_hbm, out_hbm,
               row_v, col_v, a0, a1, a2, a3, b0, b1v, b2, b3, bias_v,
               sa0, sa1, sa2, sa3, sb0, sb1, sb2, sb3,
               so0, so1, so2, so3):
    av = (a0, a1, a2, a3)
    bv = (b0, b1v, b2, b3)
    sas = (sa0, sa1, sa2, sa3)
    sbs = (sb0, sb1, sb2, sb3)
    sos = (so0, so1, so2, so3)
    c = lax.axis_index("c")
    s = lax.axis_index("s")
    wid = c * NS + s
    pltpu.sync_copy(row_hbm.at[wid], row_v)
    pltpu.sync_copy(col_hbm.at[wid], col_v)
    pltpu.sync_copy(b1_hbm, bias_v)

    def wait_pq(b):
        pltpu.make_async_copy(p_hbm.at[row_v.at[b]], av[b], sas[b]).wait()
        pltpu.make_async_copy(q_hbm.at[col_v.at[b]], bv[b], sbs[b]).wait()

    def issue_o(ch, b):
        pltpu.async_copy(av[b],
                         out_hbm.at[pl.ds((wid * NCH + ch) * CHUNK, CHUNK)],
                         sos[b])

    def wait_o(b):
        pltpu.make_async_copy(av[b], out_hbm.at[pl.ds(0, CHUNK)],
                              sos[b]).wait()

    def issue_pq(ch, b):
        pltpu.async_copy(p_hbm.at[row_v.at[ch]], av[b], sas[b])
        pltpu.async_copy(q_hbm.at[col_v.at[ch]], bv[b], sbs[b])

    for b in range(3):
        issue_pq(b, b)
    wait_pq(0)
    _eadd(av[0], bv[0], bias_v)
    issue_o(0, 0)
    issue_pq(3, 3)
    for ch in range(1, _NBUF):
        b = ch
        wait_pq(b)
        _eadd(av[b], bv[b], bias_v)
        issue_o(ch, b)
        wait_o(b - 1)
        issue_pq(ch + 3, (ch + 3) % _NBUF)

    @pl.loop(_NBUF, NCH - _NBUF, step=_NBUF)
    def _(j):
        for b in range(_NBUF):
            ch = j + b
            bo = (b + 3) % _NBUF
            wait_pq(b)
            _eadd(av[b], bv[b], bias_v)
            issue_o(ch, b)
            wait_o(bo)
            issue_pq(ch + 3, bo)

    for ch in range(NCH - _NBUF, NCH):
        b = ch % _NBUF
        bo = (b + 3) % _NBUF
        wait_pq(b)
        _eadd(av[b], bv[b], bias_v)
        issue_o(ch, b)
        wait_o(bo)
        if ch + 3 < NCH:
            issue_pq(ch + 3, bo)
    wait_o((NCH - 1) % _NBUF)


def _sc_edge(p, q, row3, col3, b1):
    k = pl.kernel(
        _edge_body,
        out_type=jax.ShapeDtypeStruct((EP, H), bf16),
        mesh=_mesh,
        scratch_types=(
            [pltpu.VMEM((NCH, CHUNK), jnp.int32)] * 2
            + [pltpu.VMEM((CHUNK, H), bf16)] * 8
            + [pltpu.VMEM((H,), bf16)]
            + [pltpu.SemaphoreType.DMA] * 12
        ),
        compiler_params=_sc_params,
    )
    return k(p, q, row3, col3, b1)


_BLK = 1024
_GRID = NP // _BLK


def _mm_body(x_ref, w_ref, root_ref, b_ref, xw_ref, rt_ref):
    r = pl.program_id(1)
    xb = x_ref[...]
    xw_ref[...] = jnp.dot(xb, w_ref[0], preferred_element_type=f32)

    @pl.when(r == 0)
    def _():
        rt_ref[...] = jnp.dot(xb, root_ref[...],
                              preferred_element_type=f32) + b_ref[...]


def _tc_mm(x, w, root, b, d_in):
    return pl.pallas_call(
        _mm_body,
        grid=(_GRID, R),
        in_specs=[
            pl.BlockSpec((_BLK, d_in), lambda i, r: (i, 0)),
            pl.BlockSpec((1, d_in, H), lambda i, r: (r, 0, 0)),
            pl.BlockSpec((d_in, H), lambda i, r: (0, 0)),
            pl.BlockSpec((1, H), lambda i, r: (0, 0)),
        ],
        out_specs=[
            pl.BlockSpec((_BLK, H), lambda i, r: (r * _GRID + i, 0)),
            pl.BlockSpec((_BLK, H), lambda i, r: (i, 0)),
        ],
        out_shape=[
            jax.ShapeDtypeStruct((R * NP, H), f32),
            jax.ShapeDtypeStruct((NP, H), f32),
        ],
    )(x, w, root, b.reshape(1, H))


def _comb_mm_body(p_ref, rt_ref, w_ref, root_ref, b_ref,
                  xn_ref, xw_ref, rtn_ref):
    r = pl.program_id(1)
    y = jnp.maximum(p_ref[0] + p_ref[1] + rt_ref[...], 0.0)
    xw_ref[...] = jnp.dot(y, w_ref[0], preferred_element_type=f32)

    @pl.when(r == 0)
    def _():
        xn_ref[...] = y
        rtn_ref[...] = jnp.dot(y, root_ref[...],
                               preferred_element_type=f32) + b_ref[...]


def _tc_comb_mm(part, rt, w, root, b):
    return pl.pallas_call(
        _comb_mm_body,
        grid=(_GRID, R),
        in_specs=[
            pl.BlockSpec((NC, _BLK, H), lambda i, r: (0, i, 0)),
            pl.BlockSpec((_BLK, H), lambda i, r: (i, 0)),
            pl.BlockSpec((1, H, H), lambda i, r: (r, 0, 0)),
            pl.BlockSpec((H, H), lambda i, r: (0, 0)),
            pl.BlockSpec((1, H), lambda i, r: (0, 0)),
        ],
        out_specs=[
            pl.BlockSpec((_BLK, H), lambda i, r: (i, 0)),
            pl.BlockSpec((_BLK, H), lambda i, r: (r * _GRID + i, 0)),
            pl.BlockSpec((_BLK, H), lambda i, r: (i, 0)),
        ],
        out_shape=[
            jax.ShapeDtypeStruct((NP, H), f32),
            jax.ShapeDtypeStruct((R * NP, H), f32),
            jax.ShapeDtypeStruct((NP, H), f32),
        ],
    )(part, rt, w, root, b.reshape(1, H))


def _final_body(p_ref, rt_ref, x1_ref, ecw1_ref,
                ncw1_ref, ncb1_ref, ncw2_ref, ncb2_ref,
                pp_ref, qq_ref, no_ref):
    x3 = p_ref[0] + p_ref[1] + rt_ref[...] + x1_ref[...]
    pp_ref[...] = jnp.dot(x3, ecw1_ref[0:H],
                          preferred_element_type=f32).astype(bf16)
    qq_ref[...] = jnp.dot(x3, ecw1_ref[H:2 * H],
                          preferred_element_type=f32).astype(bf16)
    hh = jnp.maximum(
        jnp.dot(x3, ncw1_ref[...], preferred_element_type=f32)
        + ncb1_ref[...], 0.0)
    no_ref[...] = jnp.dot(hh, ncw2_ref[...],
                          preferred_element_type=f32) + ncb2_ref[...]


def _tc_final(part, rt, x1, ec_w1, nc_w1, nc_b1, nc_w2, nc_b2):
    return pl.pallas_call(
        _final_body,
        grid=(_GRID,),
        in_specs=[
            pl.BlockSpec((NC, _BLK, H), lambda i: (0, i, 0)),
            pl.BlockSpec((_BLK, H), lambda i: (i, 0)),
            pl.BlockSpec((_BLK, H), lambda i: (i, 0)),
            pl.BlockSpec((2 * H, H), lambda i: (0, 0)),
            pl.BlockSpec((H, H // 2), lambda i: (0, 0)),
            pl.BlockSpec((1, H // 2), lambda i: (0, 0)),
            pl.BlockSpec((H // 2, 2), lambda i: (0, 0)),
            pl.BlockSpec((1, 2), lambda i: (0, 0)),
        ],
        out_specs=[
            pl.BlockSpec((_BLK, H), lambda i: (i, 0)),
            pl.BlockSpec((_BLK, H), lambda i: (i, 0)),
            pl.BlockSpec((_BLK, 2), lambda i: (i, 0)),
        ],
        out_shape=[
            jax.ShapeDtypeStruct((NP, H), bf16),
            jax.ShapeDtypeStruct((NP, H), bf16),
            jax.ShapeDtypeStruct((NP, 2), f32),
        ],
    )(part, rt, x1, ec_w1, nc_w1, nc_b1.reshape(1, H // 2),
      nc_w2, nc_b2.reshape(1, 2))


_EBLK = 1600


def _emlp_body(e1_ref, w2_ref, b2_ref, w3_ref, b3_ref, o_ref):
    e2 = jnp.maximum(
        jnp.dot(e1_ref[...], w2_ref[...], preferred_element_type=f32)
        + b2_ref[...], 0.0).astype(bf16)
    o_ref[...] = jnp.dot(e2, w3_ref[...],
                         preferred_element_type=f32) + b3_ref[...]


def _tc_emlp(e1, ec_w2, ec_b2, ec_w3, ec_b3):
    return pl.pallas_call(
        _emlp_body,
        grid=(E // _EBLK,),
        in_specs=[
            pl.BlockSpec((_EBLK, H), lambda i: (i, 0)),
            pl.BlockSpec((H, H // 2), lambda i: (0, 0)),
            pl.BlockSpec((1, H // 2), lambda i: (0, 0)),
            pl.BlockSpec((H // 2, 3), lambda i: (0, 0)),
            pl.BlockSpec((1, 3), lambda i: (0, 0)),
        ],
        out_specs=pl.BlockSpec((_EBLK, 3), lambda i: (i, 0)),
        out_shape=jax.ShapeDtypeStruct((E, 3), f32),
    )(e1, ec_w2.astype(bf16), ec_b2.reshape(1, H // 2),
      ec_w3.astype(bf16), ec_b3.reshape(1, 3))


@jax.jit
def _run(x, edge_index, edge_type,
         w1, root1, b1, w2, root2, b2, w3, root3, b3,
         nc_w1, nc_b1, nc_w2, nc_b2,
         ec_w1, ec_b1, ec_w2, ec_b2, ec_w3, ec_b3):
    src = edge_index[0]
    dst = edge_index[1]
    pad = EP - E
    ar = jnp.arange(pad, dtype=jnp.int32)
    src_p = jnp.concatenate([src, ar % NP]).astype(jnp.int32)
    col_p = jnp.concatenate([dst, ar % NP]).astype(jnp.int32)
    rel_p = jnp.concatenate([edge_type, ar % R]).astype(jnp.int32)
    seg = jnp.concatenate([dst * R + edge_type,
                           N * R + (ar % (NSEG - N * R))]).astype(jnp.int32)
    gidx = rel_p * NP + src_p

    seg_flat = seg
    gidx3 = gidx.reshape(NW, NCH, CHUNK)
    row3 = src_p.reshape(NW, NCH, CHUNK)
    col3 = col_p.reshape(NW, NCH, CHUNK)

    xpad = jnp.zeros((NP, IN), f32).at[:N].set(x)

    hist = _sc_hist(seg_flat)
    we = _sc_weights(hist, seg_flat)
    we3 = we.reshape(NW, NCH, CHUNK)

    xw1, rt1 = _tc_mm(xpad, w1, root1, b1, IN)
    part1 = _sc_layer(xw1, gidx3, col3, we3)

    x1, xw2, rt2 = _tc_comb_mm(part1, rt1, w2, root2, b2)
    part2 = _sc_layer(xw2, gidx3, col3, we3)

    _, xw3, rt3 = _tc_comb_mm(part2, rt2, w3, root3, b3)
    part3 = _sc_layer(xw3, gidx3, col3, we3)

    pp, qq, node_out = _tc_final(part3, rt3, x1, ec_w1,
                                 nc_w1, nc_b1, nc_w2, nc_b2)

    e1 = _sc_edge(pp, qq, row3, col3, ec_b1.astype(bf16))
    edge_out = _tc_emlp(e1, ec_w2, ec_b2, ec_w3, ec_b3)

    return node_out[:N], edge_out


def kernel(x, edge_index, edge_type,
           w1, root1, b1, w2, root2, b2, w3, root3, b3,
           nc_w1, nc_b1, nc_w2, nc_b2,
           ec_w1, ec_b1, ec_w2, ec_b2, ec_w3, ec_b3):
    return _run(x, edge_index, edge_type,
                w1, root1, b1, w2, root2, b2, w3, root3, b3,
                nc_w1, nc_b1, nc_w2, nc_b2,
                ec_w1, ec_b1, ec_w2, ec_b2, ec_w3, ec_b3)


# speedup vs baseline: 17.8485x; 1.0077x over previous
import dataclasses
import functools

import jax
import jax.numpy as jnp
from jax import lax
from jax.experimental import pallas as pl
from jax.experimental.pallas import tpu as pltpu
from jax.experimental.pallas import tpu_sc as plsc

N = 10000
E = 320000
R = 3
IN = 128
H = 64

NP = 10240
NSEG = NP * R
NC, NS, L = 2, 16, 16
NW = NC * NS
CHUNK = 128
NCH = 80
EPT = NCH * CHUNK
EP = NW * EPT
SPT = NSEG // NS
NPT = NP // NS

_mesh = plsc.VectorSubcoreMesh(core_axis_name="c", subcore_axis_name="s")
f32 = jnp.float32

_sc_params = pltpu.CompilerParams()
if "needs_layout_passes" in pltpu.CompilerParams.__dataclass_fields__:
    _sc_params = dataclasses.replace(_sc_params, needs_layout_passes=False)
if "use_tc_tiling_on_sc" in pltpu.CompilerParams.__dataclass_fields__:
    _sc_params = dataclasses.replace(_sc_params, use_tc_tiling_on_sc=False)


def _hist_body(seg_hbm, hist_hbm, seg_v, hist_v):
    c = lax.axis_index("c")
    s = lax.axis_index("s")
    wid = c * NS + s
    pltpu.sync_copy(seg_hbm.at[pl.ds(wid * EPT, EPT)], seg_v)
    zeros = jnp.zeros((L,), f32)
    ones = jnp.ones((L,), f32)

    @pl.loop(0, NSEG, step=L)
    def _(i):
        hist_v[pl.ds(i, L)] = zeros

    @pl.loop(0, EPT, step=L)
    def _(i):
        idx = seg_v[pl.ds(i, L)]
        plsc.addupdate_scatter(hist_v, [idx], ones)

    pltpu.sync_copy(hist_v, hist_hbm.at[wid])


def _sc_hist(seg_flat):
    k = pl.kernel(
        _hist_body,
        out_type=jax.ShapeDtypeStruct((NW, NSEG), f32),
        mesh=_mesh,
        scratch_types=[
            pltpu.VMEM((EPT,), jnp.int32),
            pltpu.VMEM((NSEG,), f32),
        ],
        compiler_params=_sc_params,
    )
    return k(seg_flat)


_WCH = 1024


def _wts_body(hist_hbm, seg_hbm, we_hbm, h_v, rcp_v, seg_v, we_v, rcp_sh):
    c = lax.axis_index("c")
    s = lax.axis_index("s")
    wid = c * NS + s
    pltpu.sync_copy(hist_hbm.at[:, pl.ds(s * SPT, SPT)], h_v)

    @pl.loop(0, SPT, step=L)
    def _(i):
        cnt = h_v[0, pl.ds(i, L)]
        for t in range(1, NW):
            cnt = cnt + h_v[t, pl.ds(i, L)]
        bid = jax.lax.iota(jnp.int32, L) + (s * SPT + i)
        r = 1.0 / jnp.maximum(cnt, 1.0)
        rcp_v[pl.ds(s * SPT + i, L)] = jnp.where(bid < N * R, r, 0.0)

    pltpu.sync_copy(rcp_v.at[pl.ds(s * SPT, SPT)],
                    rcp_sh.at[pl.ds(s * SPT, SPT)])
    plsc.subcore_barrier()
    pltpu.sync_copy(rcp_sh, rcp_v)

    @pl.loop(0, EPT // _WCH)
    def _(j):
        base = wid * EPT + j * _WCH
        pltpu.sync_copy(seg_hbm.at[pl.ds(base, _WCH)], seg_v)

        @pl.loop(0, _WCH, step=L)
        def _(i):
            idx = seg_v[pl.ds(i, L)]
            we_v[pl.ds(i, L)] = plsc.load_gather(rcp_v, [idx])

        pltpu.sync_copy(we_v, we_hbm.at[pl.ds(base, _WCH)])


def _sc_weights(hist, seg_flat):
    k = pl.kernel(
        _wts_body,
        out_type=jax.ShapeDtypeStruct((EP,), f32),
        mesh=_mesh,
        scratch_types=[
            pltpu.VMEM((NW, SPT), f32),
            pltpu.VMEM((NSEG,), f32),
            pltpu.VMEM((_WCH,), jnp.int32),
            pltpu.VMEM((_WCH,), f32),
            pltpu.VMEM_SHARED((NSEG,), f32),
        ],
        compiler_params=_sc_params,
    )
    return k(hist, seg_flat)


_NBUF = 4


def _wmul(rows, we_v, ch):
    for k in range(CHUNK // L):
        wv = we_v[ch, pl.ds(k * L, L)]
        for i in range(L):
            w = wv[i]
            r = k * L + i
            for g in range(H // L):
                sl = pl.ds(g * L, L)
                rows[r, sl] = w * rows[r, sl]


def _layer_body(xw_hbm, gidx_hbm, dst_hbm, we_hbm, out_hbm,
                gidx_v, dst_v, we_v, r0, r1, r2, r3, zb_v, acc,
                sg0, sg1, sg2, sg3, ss0, ss1, ss2, ss3):
    rows = (r0, r1, r2, r3)
    sgs = (sg0, sg1, sg2, sg3)
    sss = (ss0, ss1, ss2, ss3)
    c = lax.axis_index("c")
    s = lax.axis_index("s")
    wid = c * NS + s
    pltpu.sync_copy(gidx_hbm.at[wid], gidx_v)
    pltpu.sync_copy(dst_hbm.at[wid], dst_v)
    pltpu.sync_copy(we_hbm.at[wid], we_v)

    zeros = jnp.zeros((L,), f32)

    @pl.loop(0, CHUNK)
    def _(r):
        for g in range(H // L):
            zb_v[r, pl.ds(g * L, L)] = zeros

    @pl.loop(0, NPT, step=CHUNK)
    def _(k):
        pltpu.sync_copy(zb_v, acc.at[pl.ds(s * NPT + k, CHUNK)])

    plsc.subcore_barrier()

    def wait_g(b):
        pltpu.make_async_copy(xw_hbm.at[gidx_v.at[b]], rows[b],
                              sgs[b]).wait()

    def issue_s(ch, b):
        pltpu.async_copy(rows[b], acc.at[dst_v.at[ch]], sss[b], add=True)

    def wait_s(b):
        pltpu.make_async_copy(rows[b], acc.at[dst_v.at[0]], sss[b]).wait()

    def issue_g(ch, b):
        pltpu.async_copy(xw_hbm.at[gidx_v.at[ch]], rows[b], sgs[b])

    for b in range(3):
        issue_g(b, b)
    wait_g(0)
    _wmul(rows[0], we_v, 0)
    issue_s(0, 0)
    issue_g(3, 3)
    for ch in range(1, _NBUF):
        b = ch
        wait_g(b)
        _wmul(rows[b], we_v, ch)
        issue_s(ch, b)
        wait_s(b - 1)
        issue_g(ch + 3, (ch + 3) % _NBUF)

    @pl.loop(_NBUF, NCH - _NBUF, step=_NBUF)
    def _(j):
        for b in range(_NBUF):
            ch = j + b
            bo = (b + 3) % _NBUF
            wait_g(b)
            _wmul(rows[b], we_v, ch)
            issue_s(ch, b)
            wait_s(bo)
            issue_g(ch + 3, bo)

    for ch in range(NCH - _NBUF, NCH):
        b = ch % _NBUF
        bo = (b + 3) % _NBUF
        wait_g(b)
        _wmul(rows[b], we_v, ch)
        issue_s(ch, b)
        wait_s(bo)
        if ch + 3 < NCH:
            issue_g(ch + 3, bo)
    wait_s((NCH - 1) % _NBUF)

    plsc.subcore_barrier()

    @pl.loop(0, NPT, step=CHUNK)
    def _(k):
        pltpu.sync_copy(acc.at[pl.ds(s * NPT + k, CHUNK)],
                        out_hbm.at[c, pl.ds(s * NPT + k, CHUNK)])


def _sc_layer(xw_flat, gidx3, dst3, we3):
    k = pl.kernel(
        _layer_body,
        out_type=jax.ShapeDtypeStruct((NC, NP, H), f32),
        mesh=_mesh,
        scratch_types=[
            pltpu.VMEM((NCH, CHUNK), jnp.int32),
            pltpu.VMEM((NCH, CHUNK), jnp.int32),
            pltpu.VMEM((NCH, CHUNK), f32),
            pltpu.VMEM((CHUNK, H), f32),
            pltpu.VMEM((CHUNK, H), f32),
            pltpu.VMEM((CHUNK, H), f32),
            pltpu.VMEM((CHUNK, H), f32),
            pltpu.VMEM((CHUNK, H), f32),
            pltpu.VMEM_SHARED((NP, H), f32),
            pltpu.SemaphoreType.DMA,
            pltpu.SemaphoreType.DMA,
            pltpu.SemaphoreType.DMA,
            pltpu.SemaphoreType.DMA,
            pltpu.SemaphoreType.DMA,
            pltpu.SemaphoreType.DMA,
            pltpu.SemaphoreType.DMA,
            pltpu.SemaphoreType.DMA,
        ],
        compiler_params=_sc_params,
    )
    return k(xw_flat, gidx3, dst3, we3)


bf16 = jnp.bfloat16


def _eadd(a, b, bias_v):
    @pl.loop(0, CHUNK)
    def _(n):
        for g in range(H // L):
            sl = pl.ds(g * L, L)
            a[n, sl] = jnp.maximum(a[n, sl] + b[n, sl] + bias_v[sl], 0.0)


def _edge_body(p_hbm, q_hbm, row_hbm, col_hbm, b1_hbm, out_hbm,
               row_v, col_v, a0, a1, a2, a3, b0, b1v, b2, b3, bias_v,
               sa0, sa1, sa2, sa3, sb0, sb1, sb2, sb3,
               so0, so1, so2, so3):
    av = (a0, a1, a2, a3)
    bv = (b0, b1v, b2, b3)
    sas = (sa0, sa1, sa2, sa3)
    sbs = (sb0, sb1, sb2, sb3)
    sos = (so0, so1, so2, so3)
    c = lax.axis_index("c")
    s = lax.axis_index("s")
    wid = c * NS + s
    pltpu.sync_copy(row_hbm.at[wid], row_v)
    pltpu.sync_copy(col_hbm.at[wid], col_v)
    pltpu.sync_copy(b1_hbm, bias_v)

    def wait_pq(b):
        pltpu.make_async_copy(p_hbm.at[row_v.at[b]], av[b], sas[b]).wait()
        pltpu.make_async_copy(q_hbm.at[col_v.at[b]], bv[b], sbs[b]).wait()

    def issue_o(ch, b):
        pltpu.async_copy(av[b],
                         out_hbm.at[pl.ds((wid * NCH + ch) * CHUNK, CHUNK)],
                         sos[b])

    def wait_o(b):
        pltpu.make_async_copy(av[b], out_hbm.at[pl.ds(0, CHUNK)],
                              sos[b]).wait()

    def issue_pq(ch, b):
        pltpu.async_copy(p_hbm.at[row_v.at[ch]], av[b], sas[b])
        pltpu.async_copy(q_hbm.at[col_v.at[ch]], bv[b], sbs[b])

    for b in range(3):
        issue_pq(b, b)
    wait_pq(0)
    _eadd(av[0], bv[0], bias_v)
    issue_o(0, 0)
    issue_pq(3, 3)
    for ch in range(1, _NBUF):
        b = ch
        wait_pq(b)
        _eadd(av[b], bv[b], bias_v)
        issue_o(ch, b)
        wait_o(b - 1)
        issue_pq(ch + 3, (ch + 3) % _NBUF)

    @pl.loop(_NBUF, NCH - _NBUF, step=_NBUF)
    def _(j):
        for b in range(_NBUF):
            ch = j + b
            bo = (b + 3) % _NBUF
            wait_pq(b)
            _eadd(av[b], bv[b], bias_v)
            issue_o(ch, b)
            wait_o(bo)
            issue_pq(ch + 3, bo)

    for ch in range(NCH - _NBUF, NCH):
        b = ch % _NBUF
        bo = (b + 3) % _NBUF
        wait_pq(b)
        _eadd(av[b], bv[b], bias_v)
        issue_o(ch, b)
        wait_o(bo)
        if ch + 3 < NCH:
            issue_pq(ch + 3, bo)
    wait_o((NCH - 1) % _NBUF)


def _sc_edge(p, q, row3, col3, b1):
    k = pl.kernel(
        _edge_body,
        out_type=jax.ShapeDtypeStruct((EP, H), f32),
        mesh=_mesh,
        scratch_types=(
            [pltpu.VMEM((NCH, CHUNK), jnp.int32)] * 2
            + [pltpu.VMEM((CHUNK, H), f32)] * 8
            + [pltpu.VMEM((H,), f32)]
            + [pltpu.SemaphoreType.DMA] * 12
        ),
        compiler_params=_sc_params,
    )
    return k(p, q, row3, col3, b1)


_BLK = 1024
_GRID = NP // _BLK


def _mm_body(x_ref, w_ref, root_ref, b_ref, xw_ref, rt_ref):
    r = pl.program_id(1)
    xb = x_ref[...]
    xw_ref[...] = jnp.dot(xb, w_ref[0], preferred_element_type=f32)

    @pl.when(r == 0)
    def _():
        rt_ref[...] = jnp.dot(xb, root_ref[...],
                              preferred_element_type=f32) + b_ref[...]


def _tc_mm(x, w, root, b, d_in):
    return pl.pallas_call(
        _mm_body,
        grid=(_GRID, R),
        in_specs=[
            pl.BlockSpec((_BLK, d_in), lambda i, r: (i, 0)),
            pl.BlockSpec((1, d_in, H), lambda i, r: (r, 0, 0)),
            pl.BlockSpec((d_in, H), lambda i, r: (0, 0)),
            pl.BlockSpec((1, H), lambda i, r: (0, 0)),
        ],
        out_specs=[
            pl.BlockSpec((_BLK, H), lambda i, r: (r * _GRID + i, 0)),
            pl.BlockSpec((_BLK, H), lambda i, r: (i, 0)),
        ],
        out_shape=[
            jax.ShapeDtypeStruct((R * NP, H), f32),
            jax.ShapeDtypeStruct((NP, H), f32),
        ],
    )(x, w, root, b.reshape(1, H))


def _comb_mm_body(p_ref, rt_ref, w_ref, root_ref, b_ref,
                  xn_ref, xw_ref, rtn_ref):
    r = pl.program_id(1)
    y = jnp.maximum(p_ref[0] + p_ref[1] + rt_ref[...], 0.0)
    xw_ref[...] = jnp.dot(y, w_ref[0], preferred_element_type=f32)

    @pl.when(r == 0)
    def _():
        xn_ref[...] = y
        rtn_ref[...] = jnp.dot(y, root_ref[...],
                               preferred_element_type=f32) + b_ref[...]


def _tc_comb_mm(part, rt, w, root, b):
    return pl.pallas_call(
        _comb_mm_body,
        grid=(_GRID, R),
        in_specs=[
            pl.BlockSpec((NC, _BLK, H), lambda i, r: (0, i, 0)),
            pl.BlockSpec((_BLK, H), lambda i, r: (i, 0)),
            pl.BlockSpec((1, H, H), lambda i, r: (r, 0, 0)),
            pl.BlockSpec((H, H), lambda i, r: (0, 0)),
            pl.BlockSpec((1, H), lambda i, r: (0, 0)),
        ],
        out_specs=[
            pl.BlockSpec((_BLK, H), lambda i, r: (i, 0)),
            pl.BlockSpec((_BLK, H), lambda i, r: (r * _GRID + i, 0)),
            pl.BlockSpec((_BLK, H), lambda i, r: (i, 0)),
        ],
        out_shape=[
            jax.ShapeDtypeStruct((NP, H), f32),
            jax.ShapeDtypeStruct((R * NP, H), f32),
            jax.ShapeDtypeStruct((NP, H), f32),
        ],
    )(part, rt, w, root, b.reshape(1, H))


def _final_body(p_ref, rt_ref, x1_ref, ecw1_ref,
                ncw1_ref, ncb1_ref, ncw2_ref, ncb2_ref,
                pp_ref, qq_ref, no_ref):
    x3 = p_ref[0] + p_ref[1] + rt_ref[...] + x1_ref[...]
    pp_ref[...] = jnp.dot(x3, ecw1_ref[0:H], preferred_element_type=f32)
    qq_ref[...] = jnp.dot(x3, ecw1_ref[H:2 * H], preferred_element_type=f32)
    hh = jnp.maximum(
        jnp.dot(x3, ncw1_ref[...], preferred_element_type=f32)
        + ncb1_ref[...], 0.0)
    no_ref[...] = jnp.dot(hh, ncw2_ref[...],
                          preferred_element_type=f32) + ncb2_ref[...]


def _tc_final(part, rt, x1, ec_w1, nc_w1, nc_b1, nc_w2, nc_b2):
    return pl.pallas_call(
        _final_body,
        grid=(_GRID,),
        in_specs=[
            pl.BlockSpec((NC, _BLK, H), lambda i: (0, i, 0)),
            pl.BlockSpec((_BLK, H), lambda i: (i, 0)),
            pl.BlockSpec((_BLK, H), lambda i: (i, 0)),
            pl.BlockSpec((2 * H, H), lambda i: (0, 0)),
            pl.BlockSpec((H, H // 2), lambda i: (0, 0)),
            pl.BlockSpec((1, H // 2), lambda i: (0, 0)),
            pl.BlockSpec((H // 2, 2), lambda i: (0, 0)),
            pl.BlockSpec((1, 2), lambda i: (0, 0)),
        ],
        out_specs=[
            pl.BlockSpec((_BLK, H), lambda i: (i, 0)),
            pl.BlockSpec((_BLK, H), lambda i: (i, 0)),
            pl.BlockSpec((_BLK, 2), lambda i: (i, 0)),
        ],
        out_shape=[
            jax.ShapeDtypeStruct((NP, H), f32),
            jax.ShapeDtypeStruct((NP, H), f32),
            jax.ShapeDtypeStruct((NP, 2), f32),
        ],
    )(part, rt, x1, ec_w1, nc_w1, nc_b1.reshape(1, H // 2),
      nc_w2, nc_b2.reshape(1, 2))


_EBLK = 800


def _emlp_body(e1_ref, w2_ref, b2_ref, w3_ref, b3_ref, o_ref):
    e2 = jnp.maximum(
        jnp.dot(e1_ref[...], w2_ref[...], preferred_element_type=f32)
        + b2_ref[...], 0.0)
    o_ref[...] = jnp.dot(e2, w3_ref[...],
                         preferred_element_type=f32) + b3_ref[...]


def _tc_emlp(e1, ec_w2, ec_b2, ec_w3, ec_b3):
    e1v = e1.reshape(EP // 2, 2 * H)
    hw = H // 2
    w2bd = jnp.zeros((2 * H, H), f32)
    w2bd = w2bd.at[0:H, 0:hw].set(ec_w2).at[H:2 * H, hw:H].set(ec_w2)
    b2bd = jnp.concatenate([ec_b2, ec_b2]).reshape(1, H)
    w3bd = jnp.zeros((H, 6), f32)
    w3bd = w3bd.at[0:hw, 0:3].set(ec_w3).at[hw:H, 3:6].set(ec_w3)
    b3bd = jnp.concatenate([ec_b3, ec_b3]).reshape(1, 6)
    out = pl.pallas_call(
        _emlp_body,
        grid=(E // (2 * _EBLK),),
        in_specs=[
            pl.BlockSpec((_EBLK, 2 * H), lambda i: (i, 0)),
            pl.BlockSpec((2 * H, H), lambda i: (0, 0)),
            pl.BlockSpec((1, H), lambda i: (0, 0)),
            pl.BlockSpec((H, 6), lambda i: (0, 0)),
            pl.BlockSpec((1, 6), lambda i: (0, 0)),
        ],
        out_specs=pl.BlockSpec((_EBLK, 6), lambda i: (i, 0)),
        out_shape=jax.ShapeDtypeStruct((E // 2, 6), f32),
    )(e1v, w2bd, b2bd, w3bd, b3bd)
    return out.reshape(E, 3)


@jax.jit
def _run(x, edge_index, edge_type,
         w1, root1, b1, w2, root2, b2, w3, root3, b3,
         nc_w1, nc_b1, nc_w2, nc_b2,
         ec_w1, ec_b1, ec_w2, ec_b2, ec_w3, ec_b3):
    src = edge_index[0]
    dst = edge_index[1]
    pad = EP - E
    ar = jnp.arange(pad, dtype=jnp.int32)
    src_p = jnp.concatenate([src, ar % NP]).astype(jnp.int32)
    col_p = jnp.concatenate([dst, ar % NP]).astype(jnp.int32)
    rel_p = jnp.concatenate([edge_type, ar % R]).astype(jnp.int32)
    seg = jnp.concatenate([dst * R + edge_type,
                           N * R + (ar % (NSEG - N * R))]).astype(jnp.int32)
    gidx = rel_p * NP + src_p

    seg_flat = seg
    gidx3 = gidx.reshape(NW, NCH, CHUNK)
    row3 = src_p.reshape(NW, NCH, CHUNK)
    col3 = col_p.reshape(NW, NCH, CHUNK)

    xpad = jnp.zeros((NP, IN), f32).at[:N].set(x)

    hist = _sc_hist(seg_flat)
    we = _sc_weights(hist, seg_flat)
    we3 = we.reshape(NW, NCH, CHUNK)

    xw1, rt1 = _tc_mm(xpad, w1, root1, b1, IN)
    part1 = _sc_layer(xw1, gidx3, col3, we3)

    x1, xw2, rt2 = _tc_comb_mm(part1, rt1, w2, root2, b2)
    part2 = _sc_layer(xw2, gidx3, col3, we3)

    _, xw3, rt3 = _tc_comb_mm(part2, rt2, w3, root3, b3)
    part3 = _sc_layer(xw3, gidx3, col3, we3)

    pp, qq, node_out = _tc_final(part3, rt3, x1, ec_w1,
                                 nc_w1, nc_b1, nc_w2, nc_b2)

    e1 = _sc_edge(pp, qq, row3, col3, ec_b1)
    edge_out = _tc_emlp(e1, ec_w2, ec_b2, ec_w3, ec_b3)

    return node_out[:N], edge_out


def kernel(x, edge_index, edge_type,
           w1, root1, b1, w2, root2, b2, w3, root3, b3,
           nc_w1, nc_b1, nc_w2, nc_b2,
           ec_w1, ec_b1, ec_w2, ec_b2, ec_w3, ec_b3):
    return _run(x, edge_index, edge_type,
                w1, root1, b1, w2, root2, b2, w3, root3, b3,
                nc_w1, nc_b1, nc_w2, nc_b2,
                ec_w1, ec_b1, ec_w2, ec_b2, ec_w3, ec_b3)
